# baseline (speedup 1.0000x reference)
import functools

import jax
import jax.numpy as jnp
from jax import lax
from jax.experimental import pallas as pl
from jax.experimental.pallas import tpu as pltpu
from jax.experimental.pallas import tpu_sc as plsc

NN = 10000
NP = 10240
NE = 160000
EP = 163840
H = 256
HH = 128
NG = 256
NC, NS, CH = 2, 16, 64
EPT = EP // NS
EPW = EP // (NC * NS)
NPT = NP // NS

f32 = jnp.float32
i32 = jnp.int32


@functools.cache
def _mesh():
    return plsc.VectorSubcoreMesh(core_axis_name="c", subcore_axis_name="s",
                                  num_cores=NC, num_subcores=NS)


def _relu(v):
    return jnp.maximum(v, 0.0)


@functools.cache
def _sc_gather_x():
    @functools.partial(
        pl.kernel, mesh=_mesh(),
        out_type=[jax.ShapeDtypeStruct((EP, 16), f32)],
        scratch_types=[pltpu.VMEM((CH,), i32), pltpu.VMEM((CH,), i32),
                       pltpu.VMEM((CH, 16), f32), pltpu.VMEM((CH, 16), f32)]
        + [pltpu.SemaphoreType.DMA] * 6,
        compiler_params=pltpu.CompilerParams(use_tc_tiling_on_sc=False),
    )
    def body(x16, src, xg, idx0, idx1, row0, row1, li0, li1, q0, q1, w0, w1):
        c = lax.axis_index("c")
        s = lax.axis_index("s")
        ebase = (s * NC + c) * EPW
        idxb, rowb, li, q, w = (idx0, idx1), (row0, row1), (li0, li1), \
            (q0, q1), (w0, w1)
        nch = EPW // CH

        for b in range(2):
            pltpu.async_copy(src.at[pl.ds(ebase + b * CH, CH)], idxb[b], li[b])

        def rnd(g, carry):
            for b in range(2):
                k = g * 2 + b
                off = pl.multiple_of(ebase + k * CH, CH)
                pltpu.make_async_copy(src.at[pl.ds(off, CH)], idxb[b],
                                      li[b]).wait()

                @pl.when(k >= 2)
                def _():
                    pltpu.make_async_copy(rowb[b], xg.at[pl.ds(off, CH)],
                                          w[b]).wait()

                dg = pltpu.async_copy(x16.at[idxb[b]], rowb[b], q[b])
                dg.wait()
                pltpu.async_copy(rowb[b], xg.at[pl.ds(off, CH)], w[b])

                @pl.when(k + 2 < nch)
                def _():
                    noff = pl.multiple_of(ebase + (k + 2) * CH, CH)
                    pltpu.async_copy(src.at[pl.ds(noff, CH)], idxb[b], li[b])
            return carry

        lax.fori_loop(0, nch // 2, rnd, 0)
        for b in range(2):
            pltpu.make_async_copy(rowb[b], xg.at[pl.ds(ebase, CH)],
                                  w[b]).wait()

    return body


def _ph_scatter(rows_src, idx_src, acc, ia, ra, li, lr, sc, ebase, nch):
    nb = len(ra)
    for b in range(nb):
        off = pl.multiple_of(ebase + b * CH, CH)
        pltpu.async_copy(idx_src.at[pl.ds(off, CH)], ia[b], li[b])
        pltpu.async_copy(rows_src.at[pl.ds(off, CH)], ra[b], lr[b])

    def rnd(g, carry):
        for b in range(nb):
            k = g * nb + b
            off = pl.multiple_of(ebase + k * CH, CH)
            pltpu.make_async_copy(idx_src.at[pl.ds(off, CH)], ia[b],
                                  li[b]).wait()
            pltpu.make_async_copy(rows_src.at[pl.ds(off, CH)], ra[b],
                                  lr[b]).wait()
            pltpu.async_copy(ra[b], acc.at[ia[b]], sc, add=True).wait()

            @pl.when(k + nb < nch)
            def _():
                noff = pl.multiple_of(ebase + (k + nb) * CH, CH)
                pltpu.async_copy(idx_src.at[pl.ds(noff, CH)], ia[b], li[b])
                pltpu.async_copy(rows_src.at[pl.ds(noff, CH)], ra[b], lr[b])
        return carry

    lax.fori_loop(0, nch // nb, rnd, 0)


@functools.cache
def _sc_depth():
    @functools.partial(
        pl.kernel, mesh=_mesh(),
        out_type=[jax.ShapeDtypeStruct((NP, HH), f32) for _ in range(2)]
        + [jax.ShapeDtypeStruct((EP, HH), f32) for _ in range(4)],
        scratch_types=[pltpu.VMEM_SHARED((NP, HH), f32)]
        + [pltpu.VMEM((CH,), i32)] * 4 + [pltpu.VMEM((CH, HH), f32)] * 4
        + [pltpu.SemaphoreType.DMA] * 17,
    )
    def body(m_a, m_b, dst, src, rev, zrows,
             nm_a, nm_b, g1_a, g1_b, g2_a, g2_b,
             acc, ia0, ia1, ia2, ia3, ra0, ra1, ra2, ra3,
             li0, li1, li2, li3, lr0, lr1, lr2, lr3, sc,
             q10, q11, q20, q21, w10, w11, w20, w21):
        c = lax.axis_index("c")
        s = lax.axis_index("s")
        ebase = s * EPT
        nch = EPT // CH
        ia = (ia0, ia1, ia2, ia3)
        ra = (ra0, ra1, ra2, ra3)
        li = (li0, li1, li2, li3)
        lr = (lr0, lr1, lr2, lr3)
        qs = (q10, q11, q20, q21)
        ws = (w10, w11, w20, w21)

        def core(m_h, nm_h, g1_h, g2_h):
            pltpu.sync_copy(zrows, acc.at[pl.ds(s * NPT, NPT)])
            for b in range(2):
                off = pl.multiple_of(ebase + b * CH, CH)
                pltpu.async_copy(dst.at[pl.ds(off, CH)], ia[b], li[b])
                pltpu.async_copy(m_h.at[pl.ds(off, CH)], ra[b], lr[b])
                pltpu.async_copy(rev.at[pl.ds(off, CH)], ia[2 + b], li[2 + b])
            plsc.subcore_barrier()

            def ph_a(g, carry):
                for b in range(2):
                    k = g * 2 + b
                    off = pl.multiple_of(ebase + k * CH, CH)
                    pltpu.make_async_copy(dst.at[pl.ds(off, CH)], ia[b],
                                          li[b]).wait()
                    pltpu.make_async_copy(m_h.at[pl.ds(off, CH)], ra[b],
                                          lr[b]).wait()
                    pltpu.async_copy(ra[b], acc.at[ia[b]], sc,
                                     add=True).wait()

                    pltpu.make_async_copy(rev.at[pl.ds(off, CH)], ia[2 + b],
                                          li[2 + b]).wait()

                    @pl.when(k >= 2)
                    def _():
                        pltpu.make_async_copy(ra[2 + b],
                                              g2_h.at[pl.ds(off, CH)],
                                              ws[2 + b]).wait()

                    pltpu.async_copy(m_h.at[ia[2 + b]], ra[2 + b],
                                     qs[2 + b]).wait()
                    pltpu.async_copy(ra[2 + b], g2_h.at[pl.ds(off, CH)],
                                     ws[2 + b])

                    @pl.when(k + 2 < nch)
                    def _():
                        noff = pl.multiple_of(ebase + (k + 2) * CH, CH)
                        pltpu.async_copy(dst.at[pl.ds(noff, CH)], ia[b],
                                         li[b])
                        pltpu.async_copy(m_h.at[pl.ds(noff, CH)], ra[b],
                                         lr[b])
                        pltpu.async_copy(rev.at[pl.ds(noff, CH)], ia[2 + b],
                                         li[2 + b])
                return carry

            lax.fori_loop(0, nch // 2, ph_a, 0)
            for b in range(2):
                pltpu.make_async_copy(ra[2 + b], g2_h.at[pl.ds(ebase, CH)],
                                      ws[2 + b]).wait()
            for b in range(4):
                off = pl.multiple_of(ebase + b * CH, CH)
                pltpu.async_copy(src.at[pl.ds(off, CH)], ia[b], li[b])
            plsc.subcore_barrier()
            pltpu.sync_copy(acc.at[pl.ds(s * NPT, NPT)],
                            nm_h.at[pl.ds(s * NPT, NPT)])
            plsc.subcore_barrier()

            def ph_c(g, carry):
                for b in range(4):
                    k = g * 4 + b
                    off = pl.multiple_of(ebase + k * CH, CH)
                    pltpu.make_async_copy(src.at[pl.ds(off, CH)], ia[b],
                                          li[b]).wait()

                    @pl.when(k >= 4)
                    def _():
                        pltpu.make_async_copy(ra[b], g1_h.at[pl.ds(off, CH)],
                                              ws[b]).wait()

                    pltpu.async_copy(nm_h.at[ia[b]], ra[b], qs[b]).wait()
                    pltpu.async_copy(ra[b], g1_h.at[pl.ds(off, CH)],
                                     ws[b])

                    @pl.when(k + 4 < nch)
                    def _():
                        noff = pl.multiple_of(ebase + (k + 4) * CH, CH)
                        pltpu.async_copy(src.at[pl.ds(noff, CH)], ia[b],
                                         li[b])
                return carry

            lax.fori_loop(0, nch // 4, ph_c, 0)
            for b in range(4):
                pltpu.make_async_copy(ra[b], g1_h.at[pl.ds(ebase, CH)],
                                      ws[b]).wait()

        pl.when(c == 0)(lambda: core(m_a, nm_a, g1_a, g2_a))
        pl.when(c == 1)(lambda: core(m_b, nm_b, g1_b, g2_b))

    return body


@functools.cache
def _sc_segsum():
    @functools.partial(
        pl.kernel, mesh=_mesh(),
        out_type=[jax.ShapeDtypeStruct((NP, HH), f32) for _ in range(2)],
        scratch_types=[pltpu.VMEM_SHARED((NP, HH), f32)]
        + [pltpu.VMEM((CH,), i32)] * 4 + [pltpu.VMEM((CH, HH), f32)] * 4
        + [pltpu.SemaphoreType.DMA] * 9,
    )
    def body(h_a, h_b, dst, zrows, ns_a, ns_b,
             acc, ia0, ia1, ia2, ia3, ra0, ra1, ra2, ra3,
             li0, li1, li2, li3, lr0, lr1, lr2, lr3, sc):
        c = lax.axis_index("c")
        s = lax.axis_index("s")
        ebase = s * EPT
        ia, ra = (ia0, ia1, ia2, ia3), (ra0, ra1, ra2, ra3)
        li, lr = (li0, li1, li2, li3), (lr0, lr1, lr2, lr3)

        def core(h_h, ns_h):
            pltpu.sync_copy(zrows, acc.at[pl.ds(s * NPT, NPT)])
            plsc.subcore_barrier()
            _ph_scatter(h_h, dst, acc, ia, ra, li, lr, sc, ebase, EPT // CH)
            plsc.subcore_barrier()
            pltpu.sync_copy(acc.at[pl.ds(s * NPT, NPT)],
                            ns_h.at[pl.ds(s * NPT, NPT)])

        pl.when(c == 0)(lambda: core(h_a, ns_a))
        pl.when(c == 1)(lambda: core(h_b, ns_b))

    return body


@functools.cache
def _sc_pool():
    gpt = NG // NS

    nch = NPT // CH

    @functools.partial(
        pl.kernel, mesh=_mesh(),
        out_type=[jax.ShapeDtypeStruct((NG, HH), f32) for _ in range(2)],
        scratch_types=[pltpu.VMEM_SHARED((NG, HH), f32)]
        + [pltpu.VMEM((CH,), i32)] * 2 + [pltpu.VMEM((CH, HH), f32)] * 2
        + [pltpu.SemaphoreType.DMA] * 5,
    )
    def body(ne_a, ne_b, bat, zrows, p_a, p_b,
             acc, ia0, ia1, ra0, ra1, li0, li1, lr0, lr1, sc):
        c = lax.axis_index("c")
        s = lax.axis_index("s")
        nbase = s * NPT
        ia, ra, li, lr = (ia0, ia1), (ra0, ra1), (li0, li1), (lr0, lr1)

        def core(ne_h, p_h):
            dl = [None, None]
            dr = [None, None]
            for k in range(2):
                off = nbase + k * CH
                dl[k] = pltpu.async_copy(bat.at[pl.ds(off, CH)], ia[k], li[k])
                dr[k] = pltpu.async_copy(ne_h.at[pl.ds(off, CH)], ra[k],
                                         lr[k])
            pltpu.sync_copy(zrows.at[pl.ds(0, gpt)],
                            acc.at[pl.ds(s * gpt, gpt)])
            plsc.subcore_barrier()
            for k in range(nch):
                b = k % 2
                dl[b].wait()
                dr[b].wait()
                pltpu.async_copy(ra[b], acc.at[ia[b]], sc, add=True).wait()
                if k + 2 < nch:
                    off = nbase + (k + 2) * CH
                    dl[b] = pltpu.async_copy(bat.at[pl.ds(off, CH)], ia[b],
                                             li[b])
                    dr[b] = pltpu.async_copy(ne_h.at[pl.ds(off, CH)], ra[b],
                                             lr[b])
            plsc.subcore_barrier()
            pltpu.sync_copy(acc.at[pl.ds(s * gpt, gpt)],
                            p_h.at[pl.ds(s * gpt, gpt)])

        pl.when(c == 0)(lambda: core(ne_a, p_a))
        pl.when(c == 1)(lambda: core(ne_b, p_b))

    return body


BE = 2048


def _tc_a_body(xg, ea, wi, wm, h0a, h0b, ma, mb):
    xe = xg[...] + jnp.pad(ea[...], ((0, 0), (6, 7)))
    h0 = _relu(jnp.dot(xe, wi[...], preferred_element_type=f32))
    m = _relu(jnp.dot(h0, wm[...], preferred_element_type=f32))
    h0a[...] = h0[:, :HH]
    h0b[...] = h0[:, HH:]
    ma[...] = m[:, :HH]
    mb[...] = m[:, HH:]


def _tc_a(xg, ea, wi16, wm):
    eb = lambda i: (i, 0)
    return pl.pallas_call(
        _tc_a_body,
        grid=(EP // BE,),
        in_specs=[pl.BlockSpec((BE, 16), eb), pl.BlockSpec((BE, 3), eb),
                  pl.BlockSpec((16, H), lambda i: (0, 0)),
                  pl.BlockSpec((H, H), lambda i: (0, 0))],
        out_specs=[pl.BlockSpec((BE, HH), eb)] * 4,
        out_shape=[jax.ShapeDtypeStruct((EP, HH), f32)] * 4,
    )(xg, ea, wi16, wm)


def _tc_b_body(h0a, h0b, g1a, g1b, g2a, g2b, wm, ma, mb):
    ha = h0a[...] + g1a[...] - g2a[...]
    hb = h0b[...] + g1b[...] - g2b[...]
    h = jnp.concatenate([ha, hb], axis=1)
    m = _relu(jnp.dot(h, wm[...], preferred_element_type=f32))
    ma[...] = m[:, :HH]
    mb[...] = m[:, HH:]


def _tc_b(h0a, h0b, g1a, g1b, g2a, g2b, wm):
    eb = lambda i: (i, 0)
    return pl.pallas_call(
        _tc_b_body,
        grid=(EP // BE,),
        in_specs=[pl.BlockSpec((BE, HH), eb)] * 6
        + [pl.BlockSpec((H, H), lambda i: (0, 0))],
        out_specs=[pl.BlockSpec((BE, HH), eb)] * 2,
        out_shape=[jax.ShapeDtypeStruct((EP, HH), f32)] * 2,
    )(h0a, h0b, g1a, g1b, g2a, g2b, wm)


def _tc_c2_body(h0a, h0b, g1a, g1b, g2a, g2b, h3a, h3b):
    h3a[...] = h0a[...] + g1a[...] - g2a[...]
    h3b[...] = h0b[...] + g1b[...] - g2b[...]


def _tc_c2(h0a, h0b, g1a, g1b, g2a, g2b):
    eb = lambda i: (i, 0)
    return pl.pallas_call(
        _tc_c2_body,
        grid=(EP // BE,),
        in_specs=[pl.BlockSpec((BE, HH), eb)] * 6,
        out_specs=[pl.BlockSpec((BE, HH), eb)] * 2,
        out_shape=[jax.ShapeDtypeStruct((EP, HH), f32)] * 2,
    )(h0a, h0b, g1a, g1b, g2a, g2b)


def _tc_c_body(xp, nsa, nsb, wax, wah, nea, neb):
    ns = jnp.concatenate([nsa[...], nsb[...]], axis=1)
    ne = _relu(jnp.dot(xp[...], wax[...], preferred_element_type=f32)
               + jnp.dot(ns, wah[...], preferred_element_type=f32))
    nea[...] = ne[:, :HH]
    neb[...] = ne[:, HH:]


def _tc_c(x16, nsa, nsb, wax16, wah):
    nb = lambda i: (i, 0)
    nbk = 2048
    return pl.pallas_call(
        _tc_c_body,
        grid=(NP // nbk,),
        in_specs=[pl.BlockSpec((nbk, 16), nb), pl.BlockSpec((nbk, HH), nb),
                  pl.BlockSpec((nbk, HH), nb),
                  pl.BlockSpec((16, H), lambda i: (0, 0)),
                  pl.BlockSpec((H, H), lambda i: (0, 0))],
        out_specs=[pl.BlockSpec((nbk, HH), nb)] * 2,
        out_shape=[jax.ShapeDtypeStruct((NP, HH), f32)] * 2,
    )(x16, nsa, nsb, wax16, wah)


def _tc_d_body(pa, pb, w1, b1, w2, b2, wl, bl, out):
    p = jnp.concatenate([pa[...], pb[...]], axis=1)
    f1 = _relu(jnp.dot(p, w1[...], preferred_element_type=f32) + b1[...])
    f2 = jnp.dot(f1, w2[...], preferred_element_type=f32) + b2[...]
    out[...] = jnp.dot(f2, wl[...], preferred_element_type=f32) + bl[...]


def _tc_d(pa, pb, w1, b1, w2, b2, wl, bl):
    return pl.pallas_call(
        _tc_d_body,
        out_shape=jax.ShapeDtypeStruct((NG, 128), f32),
    )(pa, pb, w1, b1, w2, b2, wl, bl)


def kernel(x, edge_index, edge_attr, rev_edge, batch, depth,
           Wi, Wm, Wa, W1, b1, W2, b2, Wl, bl):
    src = edge_index[0].astype(i32)
    dst = edge_index[1].astype(i32)
    rev = rev_edge.astype(i32)
    bat = batch.astype(i32)

    padi = jnp.full((EP - NE,), NN, dtype=i32)
    src_p = jnp.concatenate([src, padi])
    dst_p = jnp.concatenate([dst, padi])
    rev_p = jnp.concatenate([rev, jnp.arange(NE, EP, dtype=i32)])
    ea_p = jnp.zeros((EP, 3), f32).at[:NE].set(edge_attr)
    x16 = jnp.zeros((NP, 16), f32).at[:NN, :6].set(x)
    bat_p = jnp.zeros((NP,), i32).at[:NN].set(bat)
    zrows = jnp.zeros((NPT, HH), f32)

    wi16 = jnp.zeros((16, H), f32).at[:9].set(Wi)
    wax16 = jnp.zeros((16, H), f32).at[:6].set(Wa[:6])
    wah = Wa[6:]

    (xg,) = _sc_gather_x()(x16, src_p)
    h0a, h0b, ma, mb = _tc_a(xg, ea_p, wi16, Wm)

    g1a = g1b = g2a = g2b = None
    for i in range(3):
        _, _, g1a, g1b, g2a, g2b = _sc_depth()(ma, mb, dst_p, src_p, rev_p,
                                               zrows)
        if i < 2:
            ma, mb = _tc_b(h0a, h0b, g1a, g1b, g2a, g2b, Wm)

    h3a, h3b = _tc_c2(h0a, h0b, g1a, g1b, g2a, g2b)
    nsa, nsb = _sc_segsum()(h3a, h3b, dst_p, zrows)
    nea, neb = _tc_c(x16, nsa, nsb, wax16, wah)
    pa, pb = _sc_pool()(nea, neb, bat_p, zrows)
    return _tc_d(pa, pb, W1, b1.reshape(1, -1), W2, b2.reshape(1, -1),
                 Wl, bl.reshape(1, -1))


# speedup vs baseline: 1.5328x; 1.5328x over previous
import functools

import jax
import jax.numpy as jnp
from jax import lax
from jax.experimental import pallas as pl
from jax.experimental.pallas import tpu as pltpu
from jax.experimental.pallas import tpu_sc as plsc

NN = 10000
NP = 10240
NE = 160000
EP = 163840
H = 256
HH = 128
NG = 256
NC, NS, CH = 2, 16, 64
EPT = EP // NS
EPW = EP // (NC * NS)
NPT = NP // NS

f32 = jnp.float32
i32 = jnp.int32


@functools.cache
def _mesh():
    return plsc.VectorSubcoreMesh(core_axis_name="c", subcore_axis_name="s",
                                  num_cores=NC, num_subcores=NS)


def _relu(v):
    return jnp.maximum(v, 0.0)


@functools.cache
def _sc_gather_x():
    @functools.partial(
        pl.kernel, mesh=_mesh(),
        out_type=[jax.ShapeDtypeStruct((EP, 16), f32)],
        scratch_types=[pltpu.VMEM((CH,), i32), pltpu.VMEM((CH,), i32),
                       pltpu.VMEM((CH, 16), f32), pltpu.VMEM((CH, 16), f32)]
        + [pltpu.SemaphoreType.DMA] * 6,
        compiler_params=pltpu.CompilerParams(use_tc_tiling_on_sc=False),
    )
    def body(x16, src, xg, idx0, idx1, row0, row1, li0, li1, q0, q1, w0, w1):
        c = lax.axis_index("c")
        s = lax.axis_index("s")
        ebase = (s * NC + c) * EPW
        idxb, rowb, li, q, w = (idx0, idx1), (row0, row1), (li0, li1), \
            (q0, q1), (w0, w1)
        nch = EPW // CH

        for b in range(2):
            pltpu.async_copy(src.at[pl.ds(ebase + b * CH, CH)], idxb[b], li[b])

        def rnd(g, carry):
            for b in range(2):
                k = g * 2 + b
                off = pl.multiple_of(ebase + k * CH, CH)
                pltpu.make_async_copy(src.at[pl.ds(off, CH)], idxb[b],
                                      li[b]).wait()

                @pl.when(k >= 2)
                def _():
                    pltpu.make_async_copy(rowb[b], xg.at[pl.ds(off, CH)],
                                          w[b]).wait()

                dg = pltpu.async_copy(x16.at[idxb[b]], rowb[b], q[b])
                dg.wait()
                pltpu.async_copy(rowb[b], xg.at[pl.ds(off, CH)], w[b])

                @pl.when(k + 2 < nch)
                def _():
                    noff = pl.multiple_of(ebase + (k + 2) * CH, CH)
                    pltpu.async_copy(src.at[pl.ds(noff, CH)], idxb[b], li[b])
            return carry

        lax.fori_loop(0, nch // 2, rnd, 0)
        for b in range(2):
            pltpu.make_async_copy(rowb[b], xg.at[pl.ds(ebase, CH)],
                                  w[b]).wait()

    return body


def _ph_scatter(rows_src, idx_src, acc, ia, ra, li, lr, sc, ebase, nch):
    nb = len(ra)
    for b in range(nb):
        off = pl.multiple_of(ebase + b * CH, CH)
        pltpu.async_copy(idx_src.at[pl.ds(off, CH)], ia[b], li[b])
        pltpu.async_copy(rows_src.at[pl.ds(off, CH)], ra[b], lr[b])

    def rnd(g, carry):
        for b in range(nb):
            k = g * nb + b
            off = pl.multiple_of(ebase + k * CH, CH)
            pltpu.make_async_copy(idx_src.at[pl.ds(off, CH)], ia[b],
                                  li[b]).wait()
            pltpu.make_async_copy(rows_src.at[pl.ds(off, CH)], ra[b],
                                  lr[b]).wait()
            pltpu.async_copy(ra[b], acc.at[ia[b]], sc, add=True).wait()

            @pl.when(k + nb < nch)
            def _():
                noff = pl.multiple_of(ebase + (k + nb) * CH, CH)
                pltpu.async_copy(idx_src.at[pl.ds(noff, CH)], ia[b], li[b])
                pltpu.async_copy(rows_src.at[pl.ds(noff, CH)], ra[b], lr[b])
        return carry

    lax.fori_loop(0, nch // nb, rnd, 0)


@functools.cache
def _sc_depth():
    @functools.partial(
        pl.kernel, mesh=_mesh(),
        out_type=[jax.ShapeDtypeStruct((NP, HH), f32) for _ in range(2)]
        + [jax.ShapeDtypeStruct((EP, HH), f32) for _ in range(4)],
        scratch_types=[pltpu.VMEM_SHARED((NP, HH), f32)]
        + [pltpu.VMEM((CH,), i32)] * 4 + [pltpu.VMEM((CH, HH), f32)] * 4
        + [pltpu.SemaphoreType.DMA] * 17,
    )
    def body(m_a, m_b, dst, src, rev, zrows,
             nm_a, nm_b, g1_a, g1_b, g2_a, g2_b,
             acc, ia0, ia1, ia2, ia3, ra0, ra1, ra2, ra3,
             li0, li1, li2, li3, lr0, lr1, lr2, lr3, sc,
             q10, q11, q20, q21, w10, w11, w20, w21):
        c = lax.axis_index("c")
        s = lax.axis_index("s")
        ebase = s * EPT
        nch = EPT // CH
        ia = (ia0, ia1, ia2, ia3)
        ra = (ra0, ra1, ra2, ra3)
        li = (li0, li1, li2, li3)
        lr = (lr0, lr1, lr2, lr3)
        qs = (q10, q11, q20, q21)
        ws = (w10, w11, w20, w21)

        def core(m_h, nm_h, g1_h, g2_h):
            pltpu.sync_copy(zrows, acc.at[pl.ds(s * NPT, NPT)])
            plsc.subcore_barrier()
            _ph_scatter(m_h, dst, acc, ia, ra, li, lr, sc, ebase, nch)
            for b in range(2):
                off = pl.multiple_of(ebase + b * CH, CH)
                pltpu.async_copy(src.at[pl.ds(off, CH)], ia[b], li[b])
                pltpu.async_copy(rev.at[pl.ds(off, CH)], ia[2 + b], li[2 + b])
            plsc.subcore_barrier()
            pltpu.sync_copy(acc.at[pl.ds(s * NPT, NPT)],
                            nm_h.at[pl.ds(s * NPT, NPT)])
            plsc.subcore_barrier()

            def ph_c(g, carry):
                for b in range(2):
                    k = g * 2 + b
                    off = pl.multiple_of(ebase + k * CH, CH)
                    pltpu.make_async_copy(src.at[pl.ds(off, CH)], ia[b],
                                          li[b]).wait()
                    pltpu.make_async_copy(rev.at[pl.ds(off, CH)], ia[2 + b],
                                          li[2 + b]).wait()

                    @pl.when(k >= 2)
                    def _():
                        pltpu.make_async_copy(ra[b], g1_h.at[pl.ds(off, CH)],
                                              ws[b]).wait()
                        pltpu.make_async_copy(ra[2 + b],
                                              g2_h.at[pl.ds(off, CH)],
                                              ws[2 + b]).wait()

                    d1 = pltpu.async_copy(acc.at[ia[b]], ra[b], qs[b])
                    d2 = pltpu.async_copy(m_h.at[ia[2 + b]], ra[2 + b],
                                          qs[2 + b])
                    d1.wait()
                    d2.wait()
                    pltpu.async_copy(ra[b], g1_h.at[pl.ds(off, CH)], ws[b])
                    pltpu.async_copy(ra[2 + b], g2_h.at[pl.ds(off, CH)],
                                     ws[2 + b])

                    @pl.when(k + 2 < nch)
                    def _():
                        noff = pl.multiple_of(ebase + (k + 2) * CH, CH)
                        pltpu.async_copy(src.at[pl.ds(noff, CH)], ia[b],
                                         li[b])
                        pltpu.async_copy(rev.at[pl.ds(noff, CH)], ia[2 + b],
                                         li[2 + b])
                return carry

            lax.fori_loop(0, nch // 2, ph_c, 0)
            for b in range(2):
                pltpu.make_async_copy(ra[b], g1_h.at[pl.ds(ebase, CH)],
                                      ws[b]).wait()
                pltpu.make_async_copy(ra[2 + b], g2_h.at[pl.ds(ebase, CH)],
                                      ws[2 + b]).wait()

        pl.when(c == 0)(lambda: core(m_a, nm_a, g1_a, g2_a))
        pl.when(c == 1)(lambda: core(m_b, nm_b, g1_b, g2_b))

    return body


@functools.cache
def _sc_segsum():
    @functools.partial(
        pl.kernel, mesh=_mesh(),
        out_type=[jax.ShapeDtypeStruct((NP, HH), f32) for _ in range(2)],
        scratch_types=[pltpu.VMEM_SHARED((NP, HH), f32)]
        + [pltpu.VMEM((CH,), i32)] * 4 + [pltpu.VMEM((CH, HH), f32)] * 4
        + [pltpu.SemaphoreType.DMA] * 9,
    )
    def body(h_a, h_b, dst, zrows, ns_a, ns_b,
             acc, ia0, ia1, ia2, ia3, ra0, ra1, ra2, ra3,
             li0, li1, li2, li3, lr0, lr1, lr2, lr3, sc):
        c = lax.axis_index("c")
        s = lax.axis_index("s")
        ebase = s * EPT
        ia, ra = (ia0, ia1, ia2, ia3), (ra0, ra1, ra2, ra3)
        li, lr = (li0, li1, li2, li3), (lr0, lr1, lr2, lr3)

        def core(h_h, ns_h):
            pltpu.sync_copy(zrows, acc.at[pl.ds(s * NPT, NPT)])
            plsc.subcore_barrier()
            _ph_scatter(h_h, dst, acc, ia, ra, li, lr, sc, ebase, EPT // CH)
            plsc.subcore_barrier()
            pltpu.sync_copy(acc.at[pl.ds(s * NPT, NPT)],
                            ns_h.at[pl.ds(s * NPT, NPT)])

        pl.when(c == 0)(lambda: core(h_a, ns_a))
        pl.when(c == 1)(lambda: core(h_b, ns_b))

    return body


@functools.cache
def _sc_pool():
    gpt = NG // NS

    nch = NPT // CH

    @functools.partial(
        pl.kernel, mesh=_mesh(),
        out_type=[jax.ShapeDtypeStruct((NG, HH), f32) for _ in range(2)],
        scratch_types=[pltpu.VMEM_SHARED((NG, HH), f32)]
        + [pltpu.VMEM((CH,), i32)] * 2 + [pltpu.VMEM((CH, HH), f32)] * 2
        + [pltpu.SemaphoreType.DMA] * 5,
    )
    def body(ne_a, ne_b, bat, zrows, p_a, p_b,
             acc, ia0, ia1, ra0, ra1, li0, li1, lr0, lr1, sc):
        c = lax.axis_index("c")
        s = lax.axis_index("s")
        nbase = s * NPT
        ia, ra, li, lr = (ia0, ia1), (ra0, ra1), (li0, li1), (lr0, lr1)

        def core(ne_h, p_h):
            dl = [None, None]
            dr = [None, None]
            for k in range(2):
                off = nbase + k * CH
                dl[k] = pltpu.async_copy(bat.at[pl.ds(off, CH)], ia[k], li[k])
                dr[k] = pltpu.async_copy(ne_h.at[pl.ds(off, CH)], ra[k],
                                         lr[k])
            pltpu.sync_copy(zrows.at[pl.ds(0, gpt)],
                            acc.at[pl.ds(s * gpt, gpt)])
            plsc.subcore_barrier()
            for k in range(nch):
                b = k % 2
                dl[b].wait()
                dr[b].wait()
                pltpu.async_copy(ra[b], acc.at[ia[b]], sc, add=True).wait()
                if k + 2 < nch:
                    off = nbase + (k + 2) * CH
                    dl[b] = pltpu.async_copy(bat.at[pl.ds(off, CH)], ia[b],
                                             li[b])
                    dr[b] = pltpu.async_copy(ne_h.at[pl.ds(off, CH)], ra[b],
                                             lr[b])
            plsc.subcore_barrier()
            pltpu.sync_copy(acc.at[pl.ds(s * gpt, gpt)],
                            p_h.at[pl.ds(s * gpt, gpt)])

        pl.when(c == 0)(lambda: core(ne_a, p_a))
        pl.when(c == 1)(lambda: core(ne_b, p_b))

    return body


BE = 2048


def _tc_a_body(xg, ea, wi, wm, h0a, h0b, ma, mb):
    xe = xg[...] + jnp.pad(ea[...], ((0, 0), (6, 7)))
    h0 = _relu(jnp.dot(xe, wi[...], preferred_element_type=f32))
    m = _relu(jnp.dot(h0, wm[...], preferred_element_type=f32))
    h0a[...] = h0[:, :HH]
    h0b[...] = h0[:, HH:]
    ma[...] = m[:, :HH]
    mb[...] = m[:, HH:]


def _tc_a(xg, ea, wi16, wm):
    eb = lambda i: (i, 0)
    return pl.pallas_call(
        _tc_a_body,
        grid=(EP // BE,),
        in_specs=[pl.BlockSpec((BE, 16), eb), pl.BlockSpec((BE, 3), eb),
                  pl.BlockSpec((16, H), lambda i: (0, 0)),
                  pl.BlockSpec((H, H), lambda i: (0, 0))],
        out_specs=[pl.BlockSpec((BE, HH), eb)] * 4,
        out_shape=[jax.ShapeDtypeStruct((EP, HH), f32)] * 4,
    )(xg, ea, wi16, wm)


def _tc_b_body(h0a, h0b, g1a, g1b, g2a, g2b, wm, ma, mb):
    ha = h0a[...] + g1a[...] - g2a[...]
    hb = h0b[...] + g1b[...] - g2b[...]
    h = jnp.concatenate([ha, hb], axis=1)
    m = _relu(jnp.dot(h, wm[...], preferred_element_type=f32))
    ma[...] = m[:, :HH]
    mb[...] = m[:, HH:]


def _tc_b(h0a, h0b, g1a, g1b, g2a, g2b, wm):
    eb = lambda i: (i, 0)
    return pl.pallas_call(
        _tc_b_body,
        grid=(EP // BE,),
        in_specs=[pl.BlockSpec((BE, HH), eb)] * 6
        + [pl.BlockSpec((H, H), lambda i: (0, 0))],
        out_specs=[pl.BlockSpec((BE, HH), eb)] * 2,
        out_shape=[jax.ShapeDtypeStruct((EP, HH), f32)] * 2,
    )(h0a, h0b, g1a, g1b, g2a, g2b, wm)


def _tc_c2_body(h0a, h0b, g1a, g1b, g2a, g2b, h3a, h3b):
    h3a[...] = h0a[...] + g1a[...] - g2a[...]
    h3b[...] = h0b[...] + g1b[...] - g2b[...]


def _tc_c2(h0a, h0b, g1a, g1b, g2a, g2b):
    eb = lambda i: (i, 0)
    return pl.pallas_call(
        _tc_c2_body,
        grid=(EP // BE,),
        in_specs=[pl.BlockSpec((BE, HH), eb)] * 6,
        out_specs=[pl.BlockSpec((BE, HH), eb)] * 2,
        out_shape=[jax.ShapeDtypeStruct((EP, HH), f32)] * 2,
    )(h0a, h0b, g1a, g1b, g2a, g2b)


def _tc_c_body(xp, nsa, nsb, wax, wah, nea, neb):
    ns = jnp.concatenate([nsa[...], nsb[...]], axis=1)
    ne = _relu(jnp.dot(xp[...], wax[...], preferred_element_type=f32)
               + jnp.dot(ns, wah[...], preferred_element_type=f32))
    nea[...] = ne[:, :HH]
    neb[...] = ne[:, HH:]


def _tc_c(x16, nsa, nsb, wax16, wah):
    nb = lambda i: (i, 0)
    nbk = 2048
    return pl.pallas_call(
        _tc_c_body,
        grid=(NP // nbk,),
        in_specs=[pl.BlockSpec((nbk, 16), nb), pl.BlockSpec((nbk, HH), nb),
                  pl.BlockSpec((nbk, HH), nb),
                  pl.BlockSpec((16, H), lambda i: (0, 0)),
                  pl.BlockSpec((H, H), lambda i: (0, 0))],
        out_specs=[pl.BlockSpec((nbk, HH), nb)] * 2,
        out_shape=[jax.ShapeDtypeStruct((NP, HH), f32)] * 2,
    )(x16, nsa, nsb, wax16, wah)


def _tc_d_body(pa, pb, w1, b1, w2, b2, wl, bl, out):
    p = jnp.concatenate([pa[...], pb[...]], axis=1)
    f1 = _relu(jnp.dot(p, w1[...], preferred_element_type=f32) + b1[...])
    f2 = jnp.dot(f1, w2[...], preferred_element_type=f32) + b2[...]
    out[...] = jnp.dot(f2, wl[...], preferred_element_type=f32) + bl[...]


def _tc_d(pa, pb, w1, b1, w2, b2, wl, bl):
    return pl.pallas_call(
        _tc_d_body,
        out_shape=jax.ShapeDtypeStruct((NG, 128), f32),
    )(pa, pb, w1, b1, w2, b2, wl, bl)


def kernel(x, edge_index, edge_attr, rev_edge, batch, depth,
           Wi, Wm, Wa, W1, b1, W2, b2, Wl, bl):
    src = edge_index[0].astype(i32)
    dst = edge_index[1].astype(i32)
    rev = rev_edge.astype(i32)
    bat = batch.astype(i32)

    padi = jnp.full((EP - NE,), NN, dtype=i32)
    src_p = jnp.concatenate([src, padi])
    dst_p = jnp.concatenate([dst, padi])
    rev_p = jnp.concatenate([rev, jnp.arange(NE, EP, dtype=i32)])
    ea_p = jnp.zeros((EP, 3), f32).at[:NE].set(edge_attr)
    x16 = jnp.zeros((NP, 16), f32).at[:NN, :6].set(x)
    bat_p = jnp.zeros((NP,), i32).at[:NN].set(bat)
    zrows = jnp.zeros((NPT, HH), f32)

    wi16 = jnp.zeros((16, H), f32).at[:9].set(Wi)
    wax16 = jnp.zeros((16, H), f32).at[:6].set(Wa[:6])
    wah = Wa[6:]

    (xg,) = _sc_gather_x()(x16, src_p)
    h0a, h0b, ma, mb = _tc_a(xg, ea_p, wi16, Wm)

    g1a = g1b = g2a = g2b = None
    for i in range(3):
        _, _, g1a, g1b, g2a, g2b = _sc_depth()(ma, mb, dst_p, src_p, rev_p,
                                               zrows)
        if i < 2:
            ma, mb = _tc_b(h0a, h0b, g1a, g1b, g2a, g2b, Wm)

    h3a, h3b = _tc_c2(h0a, h0b, g1a, g1b, g2a, g2b)
    nsa, nsb = _sc_segsum()(h3a, h3b, dst_p, zrows)
    nea, neb = _tc_c(x16, nsa, nsb, wax16, wah)
    pa, pb = _sc_pool()(nea, neb, bat_p, zrows)
    return _tc_d(pa, pb, W1, b1.reshape(1, -1), W2, b2.reshape(1, -1),
                 Wl, bl.reshape(1, -1))


# speedup vs baseline: 1.6245x; 1.0599x over previous
import functools

import jax
import jax.numpy as jnp
from jax import lax
from jax.experimental import pallas as pl
from jax.experimental.pallas import tpu as pltpu
from jax.experimental.pallas import tpu_sc as plsc

NN = 10000
NP = 10240
NE = 160000
EP = 163840
H = 256
HH = 128
NG = 256
NC, NS, CH = 2, 16, 64
EPT = EP // NS
EPW = EP // (NC * NS)
NPT = NP // NS

f32 = jnp.float32
i32 = jnp.int32


@functools.cache
def _mesh():
    return plsc.VectorSubcoreMesh(core_axis_name="c", subcore_axis_name="s",
                                  num_cores=NC, num_subcores=NS)


def _relu(v):
    return jnp.maximum(v, 0.0)


@functools.cache
def _sc_gather_x():
    @functools.partial(
        pl.kernel, mesh=_mesh(),
        out_type=[jax.ShapeDtypeStruct((EP, 16), f32)],
        scratch_types=[pltpu.VMEM((CH,), i32), pltpu.VMEM((CH,), i32),
                       pltpu.VMEM((CH, 16), f32), pltpu.VMEM((CH, 16), f32)]
        + [pltpu.SemaphoreType.DMA] * 6,
        compiler_params=pltpu.CompilerParams(use_tc_tiling_on_sc=False),
    )
    def body(x16, src, xg, idx0, idx1, row0, row1, li0, li1, q0, q1, w0, w1):
        c = lax.axis_index("c")
        s = lax.axis_index("s")
        ebase = (s * NC + c) * EPW
        idxb, rowb, li, q, w = (idx0, idx1), (row0, row1), (li0, li1), \
            (q0, q1), (w0, w1)
        nch = EPW // CH

        for b in range(2):
            pltpu.async_copy(src.at[pl.ds(ebase + b * CH, CH)], idxb[b], li[b])

        def rnd(g, carry):
            for b in range(2):
                k = g * 2 + b
                off = pl.multiple_of(ebase + k * CH, CH)
                pltpu.make_async_copy(src.at[pl.ds(off, CH)], idxb[b],
                                      li[b]).wait()

                @pl.when(k >= 2)
                def _():
                    pltpu.make_async_copy(rowb[b], xg.at[pl.ds(off, CH)],
                                          w[b]).wait()

                dg = pltpu.async_copy(x16.at[idxb[b]], rowb[b], q[b])
                dg.wait()
                pltpu.async_copy(rowb[b], xg.at[pl.ds(off, CH)], w[b])

                @pl.when(k + 2 < nch)
                def _():
                    noff = pl.multiple_of(ebase + (k + 2) * CH, CH)
                    pltpu.async_copy(src.at[pl.ds(noff, CH)], idxb[b], li[b])
            return carry

        lax.fori_loop(0, nch // 2, rnd, 0)
        for b in range(2):
            pltpu.make_async_copy(rowb[b], xg.at[pl.ds(ebase, CH)],
                                  w[b]).wait()

    return body


def _ph_scatter(rows_src, idx_src, acc, ia, ra, li, lr, sc, ebase, nch):
    nb = len(ra)
    for b in range(nb):
        off = pl.multiple_of(ebase + b * CH, CH)
        pltpu.async_copy(idx_src.at[pl.ds(off, CH)], ia[b], li[b])
        pltpu.async_copy(rows_src.at[pl.ds(off, CH)], ra[b], lr[b])

    def rnd(g, carry):
        for b in range(nb):
            k = g * nb + b
            off = pl.multiple_of(ebase + k * CH, CH)
            pltpu.make_async_copy(idx_src.at[pl.ds(off, CH)], ia[b],
                                  li[b]).wait()
            pltpu.make_async_copy(rows_src.at[pl.ds(off, CH)], ra[b],
                                  lr[b]).wait()
            pltpu.async_copy(ra[b], acc.at[ia[b]], sc, add=True).wait()

            @pl.when(k + nb < nch)
            def _():
                noff = pl.multiple_of(ebase + (k + nb) * CH, CH)
                pltpu.async_copy(idx_src.at[pl.ds(noff, CH)], ia[b], li[b])
                pltpu.async_copy(rows_src.at[pl.ds(noff, CH)], ra[b], lr[b])
        return carry

    lax.fori_loop(0, nch // nb, rnd, 0)


@functools.cache
def _sc_depth():
    @functools.partial(
        pl.kernel, mesh=_mesh(),
        out_type=[jax.ShapeDtypeStruct((EP, HH), f32) for _ in range(4)],
        scratch_types=[pltpu.VMEM_SHARED((NP, HH), f32)]
        + [pltpu.VMEM((CH,), i32)] * 4 + [pltpu.VMEM((CH, HH), f32)] * 4
        + [pltpu.SemaphoreType.DMA] * 17,
    )
    def body(m_a, m_b, dst, src, rev, zrows,
             g1_a, g1_b, g2_a, g2_b,
             acc, ia0, ia1, ia2, ia3, ra0, ra1, ra2, ra3,
             li0, li1, li2, li3, lr0, lr1, lr2, lr3, sc,
             q10, q11, q20, q21, w10, w11, w20, w21):
        c = lax.axis_index("c")
        s = lax.axis_index("s")
        ebase = s * EPT
        nch = EPT // CH
        ia = (ia0, ia1, ia2, ia3)
        ra = (ra0, ra1, ra2, ra3)
        li = (li0, li1, li2, li3)
        lr = (lr0, lr1, lr2, lr3)
        qs = (q10, q11, q20, q21)
        ws = (w10, w11, w20, w21)

        def core(m_h, g1_h, g2_h):
            pltpu.sync_copy(zrows, acc.at[pl.ds(s * NPT, NPT)])
            plsc.subcore_barrier()
            _ph_scatter(m_h, dst, acc, ia, ra, li, lr, sc, ebase, nch)
            for b in range(2):
                off = pl.multiple_of(ebase + b * CH, CH)
                pltpu.async_copy(src.at[pl.ds(off, CH)], ia[b], li[b])
                pltpu.async_copy(rev.at[pl.ds(off, CH)], ia[2 + b], li[2 + b])
            plsc.subcore_barrier()

            def ph_c(g, carry):
                dsc = []
                for b in range(2):
                    k = g * 2 + b
                    off = pl.multiple_of(ebase + k * CH, CH)
                    pltpu.make_async_copy(src.at[pl.ds(off, CH)], ia[b],
                                          li[b]).wait()
                    pltpu.make_async_copy(rev.at[pl.ds(off, CH)], ia[2 + b],
                                          li[2 + b]).wait()

                    @pl.when(k >= 2)
                    def _():
                        pltpu.make_async_copy(ra[b], g1_h.at[pl.ds(off, CH)],
                                              ws[b]).wait()
                        pltpu.make_async_copy(ra[2 + b],
                                              g2_h.at[pl.ds(off, CH)],
                                              ws[2 + b]).wait()

                    d2 = pltpu.async_copy(m_h.at[ia[2 + b]], ra[2 + b],
                                          qs[2 + b])
                    d1 = pltpu.async_copy(acc.at[ia[b]], ra[b], qs[b])
                    dsc.append((d1, d2))
                for b in range(2):
                    k = g * 2 + b
                    off = pl.multiple_of(ebase + k * CH, CH)
                    d1, d2 = dsc[b]
                    d1.wait()
                    d2.wait()
                    pltpu.async_copy(ra[b], g1_h.at[pl.ds(off, CH)], ws[b])
                    pltpu.async_copy(ra[2 + b], g2_h.at[pl.ds(off, CH)],
                                     ws[2 + b])

                    @pl.when(k + 2 < nch)
                    def _():
                        noff = pl.multiple_of(ebase + (k + 2) * CH, CH)
                        pltpu.async_copy(src.at[pl.ds(noff, CH)], ia[b],
                                         li[b])
                        pltpu.async_copy(rev.at[pl.ds(noff, CH)], ia[2 + b],
                                         li[2 + b])
                return carry

            lax.fori_loop(0, nch // 2, ph_c, 0)
            for b in range(2):
                pltpu.make_async_copy(ra[b], g1_h.at[pl.ds(ebase, CH)],
                                      ws[b]).wait()
                pltpu.make_async_copy(ra[2 + b], g2_h.at[pl.ds(ebase, CH)],
                                      ws[2 + b]).wait()

        pl.when(c == 0)(lambda: core(m_a, g1_a, g2_a))
        pl.when(c == 1)(lambda: core(m_b, g1_b, g2_b))

    return body


@functools.cache
def _sc_segsum():
    @functools.partial(
        pl.kernel, mesh=_mesh(),
        out_type=[jax.ShapeDtypeStruct((NP, HH), f32) for _ in range(2)],
        scratch_types=[pltpu.VMEM_SHARED((NP, HH), f32)]
        + [pltpu.VMEM((CH,), i32)] * 4 + [pltpu.VMEM((CH, HH), f32)] * 4
        + [pltpu.SemaphoreType.DMA] * 9,
    )
    def body(h_a, h_b, dst, zrows, ns_a, ns_b,
             acc, ia0, ia1, ia2, ia3, ra0, ra1, ra2, ra3,
             li0, li1, li2, li3, lr0, lr1, lr2, lr3, sc):
        c = lax.axis_index("c")
        s = lax.axis_index("s")
        ebase = s * EPT
        ia, ra = (ia0, ia1, ia2, ia3), (ra0, ra1, ra2, ra3)
        li, lr = (li0, li1, li2, li3), (lr0, lr1, lr2, lr3)

        def core(h_h, ns_h):
            pltpu.sync_copy(zrows, acc.at[pl.ds(s * NPT, NPT)])
            plsc.subcore_barrier()
            _ph_scatter(h_h, dst, acc, ia, ra, li, lr, sc, ebase, EPT // CH)
            plsc.subcore_barrier()
            pltpu.sync_copy(acc.at[pl.ds(s * NPT, NPT)],
                            ns_h.at[pl.ds(s * NPT, NPT)])

        pl.when(c == 0)(lambda: core(h_a, ns_a))
        pl.when(c == 1)(lambda: core(h_b, ns_b))

    return body


@functools.cache
def _sc_pool():
    gpt = NG // NS

    nch = NPT // CH

    @functools.partial(
        pl.kernel, mesh=_mesh(),
        out_type=[jax.ShapeDtypeStruct((NG, HH), f32) for _ in range(2)],
        scratch_types=[pltpu.VMEM_SHARED((NG, HH), f32)]
        + [pltpu.VMEM((CH,), i32)] * 2 + [pltpu.VMEM((CH, HH), f32)] * 2
        + [pltpu.SemaphoreType.DMA] * 5,
    )
    def body(ne_a, ne_b, bat, zrows, p_a, p_b,
             acc, ia0, ia1, ra0, ra1, li0, li1, lr0, lr1, sc):
        c = lax.axis_index("c")
        s = lax.axis_index("s")
        nbase = s * NPT
        ia, ra, li, lr = (ia0, ia1), (ra0, ra1), (li0, li1), (lr0, lr1)

        def core(ne_h, p_h):
            dl = [None, None]
            dr = [None, None]
            for k in range(2):
                off = nbase + k * CH
                dl[k] = pltpu.async_copy(bat.at[pl.ds(off, CH)], ia[k], li[k])
                dr[k] = pltpu.async_copy(ne_h.at[pl.ds(off, CH)], ra[k],
                                         lr[k])
            pltpu.sync_copy(zrows.at[pl.ds(0, gpt)],
                            acc.at[pl.ds(s * gpt, gpt)])
            plsc.subcore_barrier()
            for k in range(nch):
                b = k % 2
                dl[b].wait()
                dr[b].wait()
                pltpu.async_copy(ra[b], acc.at[ia[b]], sc, add=True).wait()
                if k + 2 < nch:
                    off = nbase + (k + 2) * CH
                    dl[b] = pltpu.async_copy(bat.at[pl.ds(off, CH)], ia[b],
                                             li[b])
                    dr[b] = pltpu.async_copy(ne_h.at[pl.ds(off, CH)], ra[b],
                                             lr[b])
            plsc.subcore_barrier()
            pltpu.sync_copy(acc.at[pl.ds(s * gpt, gpt)],
                            p_h.at[pl.ds(s * gpt, gpt)])

        pl.when(c == 0)(lambda: core(ne_a, p_a))
        pl.when(c == 1)(lambda: core(ne_b, p_b))

    return body


BE = 2048


def _tc_a_body(xg, ea, wi, wm, h0a, h0b, ma, mb):
    xe = xg[...] + jnp.pad(ea[...], ((0, 0), (6, 7)))
    h0 = _relu(jnp.dot(xe, wi[...], preferred_element_type=f32))
    m = _relu(jnp.dot(h0, wm[...], preferred_element_type=f32))
    h0a[...] = h0[:, :HH]
    h0b[...] = h0[:, HH:]
    ma[...] = m[:, :HH]
    mb[...] = m[:, HH:]


def _tc_a(xg, ea, wi16, wm):
    eb = lambda i: (i, 0)
    return pl.pallas_call(
        _tc_a_body,
        grid=(EP // BE,),
        in_specs=[pl.BlockSpec((BE, 16), eb), pl.BlockSpec((BE, 3), eb),
                  pl.BlockSpec((16, H), lambda i: (0, 0)),
                  pl.BlockSpec((H, H), lambda i: (0, 0))],
        out_specs=[pl.BlockSpec((BE, HH), eb)] * 4,
        out_shape=[jax.ShapeDtypeStruct((EP, HH), f32)] * 4,
    )(xg, ea, wi16, wm)


def _tc_b_body(h0a, h0b, g1a, g1b, g2a, g2b, wm, ma, mb):
    ha = h0a[...] + g1a[...] - g2a[...]
    hb = h0b[...] + g1b[...] - g2b[...]
    h = jnp.concatenate([ha, hb], axis=1)
    m = _relu(jnp.dot(h, wm[...], preferred_element_type=f32))
    ma[...] = m[:, :HH]
    mb[...] = m[:, HH:]


def _tc_b(h0a, h0b, g1a, g1b, g2a, g2b, wm):
    eb = lambda i: (i, 0)
    return pl.pallas_call(
        _tc_b_body,
        grid=(EP // BE,),
        in_specs=[pl.BlockSpec((BE, HH), eb)] * 6
        + [pl.BlockSpec((H, H), lambda i: (0, 0))],
        out_specs=[pl.BlockSpec((BE, HH), eb)] * 2,
        out_shape=[jax.ShapeDtypeStruct((EP, HH), f32)] * 2,
    )(h0a, h0b, g1a, g1b, g2a, g2b, wm)


def _tc_c2_body(h0a, h0b, g1a, g1b, g2a, g2b, h3a, h3b):
    h3a[...] = h0a[...] + g1a[...] - g2a[...]
    h3b[...] = h0b[...] + g1b[...] - g2b[...]


def _tc_c2(h0a, h0b, g1a, g1b, g2a, g2b):
    eb = lambda i: (i, 0)
    return pl.pallas_call(
        _tc_c2_body,
        grid=(EP // BE,),
        in_specs=[pl.BlockSpec((BE, HH), eb)] * 6,
        out_specs=[pl.BlockSpec((BE, HH), eb)] * 2,
        out_shape=[jax.ShapeDtypeStruct((EP, HH), f32)] * 2,
    )(h0a, h0b, g1a, g1b, g2a, g2b)


def _tc_c_body(xp, nsa, nsb, wax, wah, nea, neb):
    ns = jnp.concatenate([nsa[...], nsb[...]], axis=1)
    ne = _relu(jnp.dot(xp[...], wax[...], preferred_element_type=f32)
               + jnp.dot(ns, wah[...], preferred_element_type=f32))
    nea[...] = ne[:, :HH]
    neb[...] = ne[:, HH:]


def _tc_c(x16, nsa, nsb, wax16, wah):
    nb = lambda i: (i, 0)
    nbk = 2048
    return pl.pallas_call(
        _tc_c_body,
        grid=(NP // nbk,),
        in_specs=[pl.BlockSpec((nbk, 16), nb), pl.BlockSpec((nbk, HH), nb),
                  pl.BlockSpec((nbk, HH), nb),
                  pl.BlockSpec((16, H), lambda i: (0, 0)),
                  pl.BlockSpec((H, H), lambda i: (0, 0))],
        out_specs=[pl.BlockSpec((nbk, HH), nb)] * 2,
        out_shape=[jax.ShapeDtypeStruct((NP, HH), f32)] * 2,
    )(x16, nsa, nsb, wax16, wah)


def _tc_d_body(pa, pb, w1, b1, w2, b2, wl, bl, out):
    p = jnp.concatenate([pa[...], pb[...]], axis=1)
    f1 = _relu(jnp.dot(p, w1[...], preferred_element_type=f32) + b1[...])
    f2 = jnp.dot(f1, w2[...], preferred_element_type=f32) + b2[...]
    out[...] = jnp.dot(f2, wl[...], preferred_element_type=f32) + bl[...]


def _tc_d(pa, pb, w1, b1, w2, b2, wl, bl):
    return pl.pallas_call(
        _tc_d_body,
        out_shape=jax.ShapeDtypeStruct((NG, 128), f32),
    )(pa, pb, w1, b1, w2, b2, wl, bl)


def kernel(x, edge_index, edge_attr, rev_edge, batch, depth,
           Wi, Wm, Wa, W1, b1, W2, b2, Wl, bl):
    src = edge_index[0].astype(i32)
    dst = edge_index[1].astype(i32)
    rev = rev_edge.astype(i32)
    bat = batch.astype(i32)

    padi = jnp.full((EP - NE,), NN, dtype=i32)
    src_p = jnp.concatenate([src, padi])
    dst_p = jnp.concatenate([dst, padi])
    rev_p = jnp.concatenate([rev, jnp.arange(NE, EP, dtype=i32)])
    ea_p = jnp.zeros((EP, 3), f32).at[:NE].set(edge_attr)
    x16 = jnp.zeros((NP, 16), f32).at[:NN, :6].set(x)
    bat_p = jnp.zeros((NP,), i32).at[:NN].set(bat)
    zrows = jnp.zeros((NPT, HH), f32)

    wi16 = jnp.zeros((16, H), f32).at[:9].set(Wi)
    wax16 = jnp.zeros((16, H), f32).at[:6].set(Wa[:6])
    wah = Wa[6:]

    (xg,) = _sc_gather_x()(x16, src_p)
    h0a, h0b, ma, mb = _tc_a(xg, ea_p, wi16, Wm)

    g1a = g1b = g2a = g2b = None
    for i in range(3):
        g1a, g1b, g2a, g2b = _sc_depth()(ma, mb, dst_p, src_p, rev_p, zrows)
        if i < 2:
            ma, mb = _tc_b(h0a, h0b, g1a, g1b, g2a, g2b, Wm)

    h3a, h3b = _tc_c2(h0a, h0b, g1a, g1b, g2a, g2b)
    nsa, nsb = _sc_segsum()(h3a, h3b, dst_p, zrows)
    nea, neb = _tc_c(x16, nsa, nsb, wax16, wah)
    pa, pb = _sc_pool()(nea, neb, bat_p, zrows)
    return _tc_d(pa, pb, W1, b1.reshape(1, -1), W2, b2.reshape(1, -1),
                 Wl, bl.reshape(1, -1))


# speedup vs baseline: 1.6337x; 1.0056x over previous
import functools

import jax
import jax.numpy as jnp
from jax import lax
from jax.experimental import pallas as pl
from jax.experimental.pallas import tpu as pltpu
from jax.experimental.pallas import tpu_sc as plsc

NN = 10000
NP = 10240
NE = 160000
EP = 163840
H = 256
HH = 128
NG = 256
NC, NS, CH = 2, 16, 64
EPT = EP // NS
EPW = EP // (NC * NS)
NPT = NP // NS

f32 = jnp.float32
i32 = jnp.int32


@functools.cache
def _mesh():
    return plsc.VectorSubcoreMesh(core_axis_name="c", subcore_axis_name="s",
                                  num_cores=NC, num_subcores=NS)


def _relu(v):
    return jnp.maximum(v, 0.0)


@functools.cache
def _sc_gather_x():
    @functools.partial(
        pl.kernel, mesh=_mesh(),
        out_type=[jax.ShapeDtypeStruct((EP, 16), f32)],
        scratch_types=[pltpu.VMEM((CH,), i32), pltpu.VMEM((CH,), i32),
                       pltpu.VMEM((CH, 16), f32), pltpu.VMEM((CH, 16), f32)]
        + [pltpu.SemaphoreType.DMA] * 6,
        compiler_params=pltpu.CompilerParams(use_tc_tiling_on_sc=False),
    )
    def body(x16, src, xg, idx0, idx1, row0, row1, li0, li1, q0, q1, w0, w1):
        c = lax.axis_index("c")
        s = lax.axis_index("s")
        ebase = (s * NC + c) * EPW
        idxb, rowb, li, q, w = (idx0, idx1), (row0, row1), (li0, li1), \
            (q0, q1), (w0, w1)
        nch = EPW // CH

        for b in range(2):
            pltpu.async_copy(src.at[pl.ds(ebase + b * CH, CH)], idxb[b], li[b])

        def rnd(g, carry):
            dg = []
            for b in range(2):
                k = g * 2 + b
                off = pl.multiple_of(ebase + k * CH, CH)
                pltpu.make_async_copy(src.at[pl.ds(off, CH)], idxb[b],
                                      li[b]).wait()

                @pl.when(k >= 2)
                def _():
                    pltpu.make_async_copy(rowb[b], xg.at[pl.ds(off, CH)],
                                          w[b]).wait()

                dg.append(pltpu.async_copy(x16.at[idxb[b]], rowb[b], q[b]))
            for b in range(2):
                k = g * 2 + b
                off = pl.multiple_of(ebase + k * CH, CH)
                dg[b].wait()
                pltpu.async_copy(rowb[b], xg.at[pl.ds(off, CH)], w[b])

                @pl.when(k + 2 < nch)
                def _():
                    noff = pl.multiple_of(ebase + (k + 2) * CH, CH)
                    pltpu.async_copy(src.at[pl.ds(noff, CH)], idxb[b], li[b])
            return carry

        lax.fori_loop(0, nch // 2, rnd, 0)
        for b in range(2):
            pltpu.make_async_copy(rowb[b], xg.at[pl.ds(ebase, CH)],
                                  w[b]).wait()

    return body


def _ph_scatter(rows_src, idx_src, acc, ia, ra, li, lr, sc, ebase, nch):
    nb = len(ra)
    for b in range(nb):
        off = pl.multiple_of(ebase + b * CH, CH)
        pltpu.async_copy(idx_src.at[pl.ds(off, CH)], ia[b], li[b])
        pltpu.async_copy(rows_src.at[pl.ds(off, CH)], ra[b], lr[b])

    def rnd(g, carry):
        for b in range(nb):
            k = g * nb + b
            off = pl.multiple_of(ebase + k * CH, CH)
            pltpu.make_async_copy(idx_src.at[pl.ds(off, CH)], ia[b],
                                  li[b]).wait()
            pltpu.make_async_copy(rows_src.at[pl.ds(off, CH)], ra[b],
                                  lr[b]).wait()
            pltpu.async_copy(ra[b], acc.at[ia[b]], sc, add=True).wait()

            @pl.when(k + nb < nch)
            def _():
                noff = pl.multiple_of(ebase + (k + nb) * CH, CH)
                pltpu.async_copy(idx_src.at[pl.ds(noff, CH)], ia[b], li[b])
                pltpu.async_copy(rows_src.at[pl.ds(noff, CH)], ra[b], lr[b])
        return carry

    lax.fori_loop(0, nch // nb, rnd, 0)


@functools.cache
def _sc_depth():
    @functools.partial(
        pl.kernel, mesh=_mesh(),
        out_type=[jax.ShapeDtypeStruct((EP, HH), f32) for _ in range(4)],
        scratch_types=[pltpu.VMEM_SHARED((NP, HH), f32)]
        + [pltpu.VMEM((CH,), i32)] * 4 + [pltpu.VMEM((CH, HH), f32)] * 4
        + [pltpu.SemaphoreType.DMA] * 17,
    )
    def body(m_a, m_b, dst, src, rev, zrows,
             g1_a, g1_b, g2_a, g2_b,
             acc, ia0, ia1, ia2, ia3, ra0, ra1, ra2, ra3,
             li0, li1, li2, li3, lr0, lr1, lr2, lr3, sc,
             q10, q11, q20, q21, w10, w11, w20, w21):
        c = lax.axis_index("c")
        s = lax.axis_index("s")
        ebase = s * EPT
        nch = EPT // CH
        ia = (ia0, ia1, ia2, ia3)
        ra = (ra0, ra1, ra2, ra3)
        li = (li0, li1, li2, li3)
        lr = (lr0, lr1, lr2, lr3)
        qs = (q10, q11, q20, q21)
        ws = (w10, w11, w20, w21)

        def core(m_h, g1_h, g2_h):
            pltpu.sync_copy(zrows, acc.at[pl.ds(s * NPT, NPT)])
            plsc.subcore_barrier()
            _ph_scatter(m_h, dst, acc, ia, ra, li, lr, sc, ebase, nch)
            for b in range(2):
                off = pl.multiple_of(ebase + b * CH, CH)
                pltpu.async_copy(src.at[pl.ds(off, CH)], ia[b], li[b])
                pltpu.async_copy(rev.at[pl.ds(off, CH)], ia[2 + b], li[2 + b])
            plsc.subcore_barrier()

            def ph_c(g, carry):
                dsc = []
                for b in range(2):
                    k = g * 2 + b
                    off = pl.multiple_of(ebase + k * CH, CH)
                    pltpu.make_async_copy(src.at[pl.ds(off, CH)], ia[b],
                                          li[b]).wait()
                    pltpu.make_async_copy(rev.at[pl.ds(off, CH)], ia[2 + b],
                                          li[2 + b]).wait()

                    @pl.when(k >= 2)
                    def _():
                        pltpu.make_async_copy(ra[b], g1_h.at[pl.ds(off, CH)],
                                              ws[b]).wait()
                        pltpu.make_async_copy(ra[2 + b],
                                              g2_h.at[pl.ds(off, CH)],
                                              ws[2 + b]).wait()

                    d2 = pltpu.async_copy(m_h.at[ia[2 + b]], ra[2 + b],
                                          qs[2 + b])
                    d1 = pltpu.async_copy(acc.at[ia[b]], ra[b], qs[b])
                    dsc.append((d1, d2))
                for b in range(2):
                    k = g * 2 + b
                    off = pl.multiple_of(ebase + k * CH, CH)
                    d1, d2 = dsc[b]
                    d1.wait()
                    d2.wait()
                    pltpu.async_copy(ra[b], g1_h.at[pl.ds(off, CH)], ws[b])
                    pltpu.async_copy(ra[2 + b], g2_h.at[pl.ds(off, CH)],
                                     ws[2 + b])

                    @pl.when(k + 2 < nch)
                    def _():
                        noff = pl.multiple_of(ebase + (k + 2) * CH, CH)
                        pltpu.async_copy(src.at[pl.ds(noff, CH)], ia[b],
                                         li[b])
                        pltpu.async_copy(rev.at[pl.ds(noff, CH)], ia[2 + b],
                                         li[2 + b])
                return carry

            lax.fori_loop(0, nch // 2, ph_c, 0)
            for b in range(2):
                pltpu.make_async_copy(ra[b], g1_h.at[pl.ds(ebase, CH)],
                                      ws[b]).wait()
                pltpu.make_async_copy(ra[2 + b], g2_h.at[pl.ds(ebase, CH)],
                                      ws[2 + b]).wait()

        pl.when(c == 0)(lambda: core(m_a, g1_a, g2_a))
        pl.when(c == 1)(lambda: core(m_b, g1_b, g2_b))

    return body


@functools.cache
def _sc_segsum():
    @functools.partial(
        pl.kernel, mesh=_mesh(),
        out_type=[jax.ShapeDtypeStruct((NP, HH), f32) for _ in range(2)],
        scratch_types=[pltpu.VMEM_SHARED((NP, HH), f32)]
        + [pltpu.VMEM((CH,), i32)] * 4 + [pltpu.VMEM((CH, HH), f32)] * 4
        + [pltpu.SemaphoreType.DMA] * 9,
    )
    def body(h_a, h_b, dst, zrows, ns_a, ns_b,
             acc, ia0, ia1, ia2, ia3, ra0, ra1, ra2, ra3,
             li0, li1, li2, li3, lr0, lr1, lr2, lr3, sc):
        c = lax.axis_index("c")
        s = lax.axis_index("s")
        ebase = s * EPT
        ia, ra = (ia0, ia1, ia2, ia3), (ra0, ra1, ra2, ra3)
        li, lr = (li0, li1, li2, li3), (lr0, lr1, lr2, lr3)

        def core(h_h, ns_h):
            pltpu.sync_copy(zrows, acc.at[pl.ds(s * NPT, NPT)])
            plsc.subcore_barrier()
            _ph_scatter(h_h, dst, acc, ia, ra, li, lr, sc, ebase, EPT // CH)
            plsc.subcore_barrier()
            pltpu.sync_copy(acc.at[pl.ds(s * NPT, NPT)],
                            ns_h.at[pl.ds(s * NPT, NPT)])

        pl.when(c == 0)(lambda: core(h_a, ns_a))
        pl.when(c == 1)(lambda: core(h_b, ns_b))

    return body


@functools.cache
def _sc_pool():
    gpt = NG // NS

    nch = NPT // CH

    @functools.partial(
        pl.kernel, mesh=_mesh(),
        out_type=[jax.ShapeDtypeStruct((NG, HH), f32) for _ in range(2)],
        scratch_types=[pltpu.VMEM_SHARED((NG, HH), f32)]
        + [pltpu.VMEM((CH,), i32)] * 2 + [pltpu.VMEM((CH, HH), f32)] * 2
        + [pltpu.SemaphoreType.DMA] * 5,
    )
    def body(ne_a, ne_b, bat, zrows, p_a, p_b,
             acc, ia0, ia1, ra0, ra1, li0, li1, lr0, lr1, sc):
        c = lax.axis_index("c")
        s = lax.axis_index("s")
        nbase = s * NPT
        ia, ra, li, lr = (ia0, ia1), (ra0, ra1), (li0, li1), (lr0, lr1)

        def core(ne_h, p_h):
            dl = [None, None]
            dr = [None, None]
            for k in range(2):
                off = nbase + k * CH
                dl[k] = pltpu.async_copy(bat.at[pl.ds(off, CH)], ia[k], li[k])
                dr[k] = pltpu.async_copy(ne_h.at[pl.ds(off, CH)], ra[k],
                                         lr[k])
            pltpu.sync_copy(zrows.at[pl.ds(0, gpt)],
                            acc.at[pl.ds(s * gpt, gpt)])
            plsc.subcore_barrier()
            for k in range(nch):
                b = k % 2
                dl[b].wait()
                dr[b].wait()
                pltpu.async_copy(ra[b], acc.at[ia[b]], sc, add=True).wait()
                if k + 2 < nch:
                    off = nbase + (k + 2) * CH
                    dl[b] = pltpu.async_copy(bat.at[pl.ds(off, CH)], ia[b],
                                             li[b])
                    dr[b] = pltpu.async_copy(ne_h.at[pl.ds(off, CH)], ra[b],
                                             lr[b])
            plsc.subcore_barrier()
            pltpu.sync_copy(acc.at[pl.ds(s * gpt, gpt)],
                            p_h.at[pl.ds(s * gpt, gpt)])

        pl.when(c == 0)(lambda: core(ne_a, p_a))
        pl.when(c == 1)(lambda: core(ne_b, p_b))

    return body


BE = 2048


bf16 = jnp.bfloat16


def _tc_a_body(xg, ea, wi, wm, h0a, h0b, ma, mb):
    xe = xg[...] + jnp.pad(ea[...], ((0, 0), (6, 7)))
    h0 = _relu(jnp.dot(xe, wi[...], preferred_element_type=f32))
    m = _relu(jnp.dot(h0.astype(bf16), wm[...].astype(bf16),
                      preferred_element_type=f32))
    h0a[...] = h0[:, :HH]
    h0b[...] = h0[:, HH:]
    ma[...] = m[:, :HH]
    mb[...] = m[:, HH:]


def _tc_a(xg, ea, wi16, wm):
    eb = lambda i: (i, 0)
    return pl.pallas_call(
        _tc_a_body,
        grid=(EP // BE,),
        in_specs=[pl.BlockSpec((BE, 16), eb), pl.BlockSpec((BE, 3), eb),
                  pl.BlockSpec((16, H), lambda i: (0, 0)),
                  pl.BlockSpec((H, H), lambda i: (0, 0))],
        out_specs=[pl.BlockSpec((BE, HH), eb)] * 4,
        out_shape=[jax.ShapeDtypeStruct((EP, HH), f32)] * 4,
    )(xg, ea, wi16, wm)


def _tc_b_body(h0a, h0b, g1a, g1b, g2a, g2b, wm, ma, mb):
    ha = h0a[...] + g1a[...] - g2a[...]
    hb = h0b[...] + g1b[...] - g2b[...]
    h = jnp.concatenate([ha, hb], axis=1)
    m = _relu(jnp.dot(h.astype(bf16), wm[...].astype(bf16),
                      preferred_element_type=f32))
    ma[...] = m[:, :HH]
    mb[...] = m[:, HH:]


def _tc_b(h0a, h0b, g1a, g1b, g2a, g2b, wm):
    eb = lambda i: (i, 0)
    return pl.pallas_call(
        _tc_b_body,
        grid=(EP // BE,),
        in_specs=[pl.BlockSpec((BE, HH), eb)] * 6
        + [pl.BlockSpec((H, H), lambda i: (0, 0))],
        out_specs=[pl.BlockSpec((BE, HH), eb)] * 2,
        out_shape=[jax.ShapeDtypeStruct((EP, HH), f32)] * 2,
    )(h0a, h0b, g1a, g1b, g2a, g2b, wm)


def _tc_c2_body(h0a, h0b, g1a, g1b, g2a, g2b, h3a, h3b):
    h3a[...] = h0a[...] + g1a[...] - g2a[...]
    h3b[...] = h0b[...] + g1b[...] - g2b[...]


def _tc_c2(h0a, h0b, g1a, g1b, g2a, g2b):
    eb = lambda i: (i, 0)
    return pl.pallas_call(
        _tc_c2_body,
        grid=(EP // BE,),
        in_specs=[pl.BlockSpec((BE, HH), eb)] * 6,
        out_specs=[pl.BlockSpec((BE, HH), eb)] * 2,
        out_shape=[jax.ShapeDtypeStruct((EP, HH), f32)] * 2,
    )(h0a, h0b, g1a, g1b, g2a, g2b)


def _tc_c_body(xp, nsa, nsb, wax, wah, nea, neb):
    ns = jnp.concatenate([nsa[...], nsb[...]], axis=1)
    ne = _relu(jnp.dot(xp[...], wax[...], preferred_element_type=f32)
               + jnp.dot(ns, wah[...], preferred_element_type=f32))
    nea[...] = ne[:, :HH]
    neb[...] = ne[:, HH:]


def _tc_c(x16, nsa, nsb, wax16, wah):
    nb = lambda i: (i, 0)
    nbk = 2048
    return pl.pallas_call(
        _tc_c_body,
        grid=(NP // nbk,),
        in_specs=[pl.BlockSpec((nbk, 16), nb), pl.BlockSpec((nbk, HH), nb),
                  pl.BlockSpec((nbk, HH), nb),
                  pl.BlockSpec((16, H), lambda i: (0, 0)),
                  pl.BlockSpec((H, H), lambda i: (0, 0))],
        out_specs=[pl.BlockSpec((nbk, HH), nb)] * 2,
        out_shape=[jax.ShapeDtypeStruct((NP, HH), f32)] * 2,
    )(x16, nsa, nsb, wax16, wah)


def _tc_d_body(pa, pb, w1, b1, w2, b2, wl, bl, out):
    p = jnp.concatenate([pa[...], pb[...]], axis=1)
    f1 = _relu(jnp.dot(p, w1[...], preferred_element_type=f32) + b1[...])
    f2 = jnp.dot(f1, w2[...], preferred_element_type=f32) + b2[...]
    out[...] = jnp.dot(f2, wl[...], preferred_element_type=f32) + bl[...]


def _tc_d(pa, pb, w1, b1, w2, b2, wl, bl):
    return pl.pallas_call(
        _tc_d_body,
        out_shape=jax.ShapeDtypeStruct((NG, 128), f32),
    )(pa, pb, w1, b1, w2, b2, wl, bl)


def kernel(x, edge_index, edge_attr, rev_edge, batch, depth,
           Wi, Wm, Wa, W1, b1, W2, b2, Wl, bl):
    src = edge_index[0].astype(i32)
    dst = edge_index[1].astype(i32)
    rev = rev_edge.astype(i32)
    bat = batch.astype(i32)

    padi = jnp.full((EP - NE,), NN, dtype=i32)
    src_p = jnp.concatenate([src, padi])
    dst_p = jnp.concatenate([dst, padi])
    rev_p = jnp.concatenate([rev, jnp.arange(NE, EP, dtype=i32)])
    ea_p = jnp.zeros((EP, 3), f32).at[:NE].set(edge_attr)
    x16 = jnp.zeros((NP, 16), f32).at[:NN, :6].set(x)
    bat_p = jnp.zeros((NP,), i32).at[:NN].set(bat)
    zrows = jnp.zeros((NPT, HH), f32)

    wi16 = jnp.zeros((16, H), f32).at[:9].set(Wi)
    wax16 = jnp.zeros((16, H), f32).at[:6].set(Wa[:6])
    wah = Wa[6:]

    (xg,) = _sc_gather_x()(x16, src_p)
    h0a, h0b, ma, mb = _tc_a(xg, ea_p, wi16, Wm)

    g1a = g1b = g2a = g2b = None
    for i in range(3):
        g1a, g1b, g2a, g2b = _sc_depth()(ma, mb, dst_p, src_p, rev_p, zrows)
        if i < 2:
            ma, mb = _tc_b(h0a, h0b, g1a, g1b, g2a, g2b, Wm)

    h3a, h3b = _tc_c2(h0a, h0b, g1a, g1b, g2a, g2b)
    nsa, nsb = _sc_segsum()(h3a, h3b, dst_p, zrows)
    nea, neb = _tc_c(x16, nsa, nsb, wax16, wah)
    pa, pb = _sc_pool()(nea, neb, bat_p, zrows)
    return _tc_d(pa, pb, W1, b1.reshape(1, -1), W2, b2.reshape(1, -1),
                 Wl, bl.reshape(1, -1))


# speedup vs baseline: 1.7098x; 1.0466x over previous
import functools

import jax
import jax.numpy as jnp
from jax import lax
from jax.experimental import pallas as pl
from jax.experimental.pallas import tpu as pltpu
from jax.experimental.pallas import tpu_sc as plsc

NN = 10000
NP = 10240
NE = 160000
EP = 163840
H = 256
HH = 128
NG = 256
NC, NS, CH = 2, 16, 64
EPT = EP // NS
EPW = EP // (NC * NS)
NPT = NP // NS

f32 = jnp.float32
i32 = jnp.int32


@functools.cache
def _mesh():
    return plsc.VectorSubcoreMesh(core_axis_name="c", subcore_axis_name="s",
                                  num_cores=NC, num_subcores=NS)


def _relu(v):
    return jnp.maximum(v, 0.0)


@functools.cache
def _sc_gather_x():
    @functools.partial(
        pl.kernel, mesh=_mesh(),
        out_type=[jax.ShapeDtypeStruct((EP, 16), f32)],
        scratch_types=[pltpu.VMEM((CH,), i32), pltpu.VMEM((CH,), i32),
                       pltpu.VMEM((CH, 16), f32), pltpu.VMEM((CH, 16), f32)]
        + [pltpu.SemaphoreType.DMA] * 6,
        compiler_params=pltpu.CompilerParams(use_tc_tiling_on_sc=False),
    )
    def body(x16, src, xg, idx0, idx1, row0, row1, li0, li1, q0, q1, w0, w1):
        c = lax.axis_index("c")
        s = lax.axis_index("s")
        ebase = (s * NC + c) * EPW
        idxb, rowb, li, q, w = (idx0, idx1), (row0, row1), (li0, li1), \
            (q0, q1), (w0, w1)
        nch = EPW // CH

        for b in range(2):
            pltpu.async_copy(src.at[pl.ds(ebase + b * CH, CH)], idxb[b], li[b])

        def rnd(g, carry):
            dg = []
            for b in range(2):
                k = g * 2 + b
                off = pl.multiple_of(ebase + k * CH, CH)
                pltpu.make_async_copy(src.at[pl.ds(off, CH)], idxb[b],
                                      li[b]).wait()

                @pl.when(k >= 2)
                def _():
                    pltpu.make_async_copy(rowb[b], xg.at[pl.ds(off, CH)],
                                          w[b]).wait()

                dg.append(pltpu.async_copy(x16.at[idxb[b]], rowb[b], q[b]))
            for b in range(2):
                k = g * 2 + b
                off = pl.multiple_of(ebase + k * CH, CH)
                dg[b].wait()
                pltpu.async_copy(rowb[b], xg.at[pl.ds(off, CH)], w[b])

                @pl.when(k + 2 < nch)
                def _():
                    noff = pl.multiple_of(ebase + (k + 2) * CH, CH)
                    pltpu.async_copy(src.at[pl.ds(noff, CH)], idxb[b], li[b])
            return carry

        lax.fori_loop(0, nch // 2, rnd, 0)
        for b in range(2):
            pltpu.make_async_copy(rowb[b], xg.at[pl.ds(ebase, CH)],
                                  w[b]).wait()

    return body


def _ph_scatter(rows_src, idx_src, acc, ia, ra, li, lr, sc, ebase, nch):
    nb = len(ra)
    for b in range(nb):
        off = pl.multiple_of(ebase + b * CH, CH)
        pltpu.async_copy(idx_src.at[pl.ds(off, CH)], ia[b], li[b])
        pltpu.async_copy(rows_src.at[pl.ds(off, CH)], ra[b], lr[b])

    def rnd(g, carry):
        for half in range(2):
            dsc = []
            for j in range(2):
                b = half * 2 + j
                k = g * 4 + b
                off = pl.multiple_of(ebase + k * CH, CH)
                pltpu.make_async_copy(idx_src.at[pl.ds(off, CH)], ia[b],
                                      li[b]).wait()
                pltpu.make_async_copy(rows_src.at[pl.ds(off, CH)], ra[b],
                                      lr[b]).wait()
                dsc.append(pltpu.async_copy(ra[b], acc.at[ia[b]], sc,
                                            add=True))
            for j in range(2):
                b = half * 2 + j
                k = g * 4 + b
                dsc[j].wait()

                @pl.when(k + nb < nch)
                def _():
                    noff = pl.multiple_of(ebase + (k + nb) * CH, CH)
                    pltpu.async_copy(idx_src.at[pl.ds(noff, CH)], ia[b],
                                     li[b])
                    pltpu.async_copy(rows_src.at[pl.ds(noff, CH)], ra[b],
                                     lr[b])
        return carry

    lax.fori_loop(0, nch // 4, rnd, 0)


@functools.cache
def _sc_depth():
    @functools.partial(
        pl.kernel, mesh=_mesh(),
        out_type=[jax.ShapeDtypeStruct((EP, HH), f32) for _ in range(4)],
        scratch_types=[pltpu.VMEM_SHARED((NP, HH), f32)]
        + [pltpu.VMEM((CH,), i32)] * 4 + [pltpu.VMEM((CH, HH), f32)] * 4
        + [pltpu.SemaphoreType.DMA] * 17,
    )
    def body(m_a, m_b, dst, src, rev, zrows,
             g1_a, g1_b, g2_a, g2_b,
             acc, ia0, ia1, ia2, ia3, ra0, ra1, ra2, ra3,
             li0, li1, li2, li3, lr0, lr1, lr2, lr3, sc,
             q10, q11, q20, q21, w10, w11, w20, w21):
        c = lax.axis_index("c")
        s = lax.axis_index("s")
        ebase = s * EPT
        nch = EPT // CH
        ia = (ia0, ia1, ia2, ia3)
        ra = (ra0, ra1, ra2, ra3)
        li = (li0, li1, li2, li3)
        lr = (lr0, lr1, lr2, lr3)
        qs = (q10, q11, q20, q21)
        ws = (w10, w11, w20, w21)

        def core(m_h, g1_h, g2_h):
            pltpu.sync_copy(zrows, acc.at[pl.ds(s * NPT, NPT)])
            plsc.subcore_barrier()
            _ph_scatter(m_h, dst, acc, ia, ra, li, lr, sc, ebase, nch)
            for b in range(2):
                off = pl.multiple_of(ebase + b * CH, CH)
                pltpu.async_copy(src.at[pl.ds(off, CH)], ia[b], li[b])
                pltpu.async_copy(rev.at[pl.ds(off, CH)], ia[2 + b], li[2 + b])
            plsc.subcore_barrier()

            def ph_c(g, carry):
                dsc = []
                for b in range(2):
                    k = g * 2 + b
                    off = pl.multiple_of(ebase + k * CH, CH)
                    pltpu.make_async_copy(src.at[pl.ds(off, CH)], ia[b],
                                          li[b]).wait()
                    pltpu.make_async_copy(rev.at[pl.ds(off, CH)], ia[2 + b],
                                          li[2 + b]).wait()

                    @pl.when(k >= 2)
                    def _():
                        pltpu.make_async_copy(ra[b], g1_h.at[pl.ds(off, CH)],
                                              ws[b]).wait()
                        pltpu.make_async_copy(ra[2 + b],
                                              g2_h.at[pl.ds(off, CH)],
                                              ws[2 + b]).wait()

                    d2 = pltpu.async_copy(m_h.at[ia[2 + b]], ra[2 + b],
                                          qs[2 + b])
                    d1 = pltpu.async_copy(acc.at[ia[b]], ra[b], qs[b])
                    dsc.append((d1, d2))
                for b in range(2):
                    k = g * 2 + b
                    off = pl.multiple_of(ebase + k * CH, CH)
                    d1, d2 = dsc[b]
                    d1.wait()
                    d2.wait()
                    pltpu.async_copy(ra[b], g1_h.at[pl.ds(off, CH)], ws[b])
                    pltpu.async_copy(ra[2 + b], g2_h.at[pl.ds(off, CH)],
                                     ws[2 + b])

                    @pl.when(k + 2 < nch)
                    def _():
                        noff = pl.multiple_of(ebase + (k + 2) * CH, CH)
                        pltpu.async_copy(src.at[pl.ds(noff, CH)], ia[b],
                                         li[b])
                        pltpu.async_copy(rev.at[pl.ds(noff, CH)], ia[2 + b],
                                         li[2 + b])
                return carry

            lax.fori_loop(0, nch // 2, ph_c, 0)
            for b in range(2):
                pltpu.make_async_copy(ra[b], g1_h.at[pl.ds(ebase, CH)],
                                      ws[b]).wait()
                pltpu.make_async_copy(ra[2 + b], g2_h.at[pl.ds(ebase, CH)],
                                      ws[2 + b]).wait()

        pl.when(c == 0)(lambda: core(m_a, g1_a, g2_a))
        pl.when(c == 1)(lambda: core(m_b, g1_b, g2_b))

    return body


@functools.cache
def _sc_segsum():
    @functools.partial(
        pl.kernel, mesh=_mesh(),
        out_type=[jax.ShapeDtypeStruct((NP, HH), f32) for _ in range(2)],
        scratch_types=[pltpu.VMEM_SHARED((NP, HH), f32)]
        + [pltpu.VMEM((CH,), i32)] * 4 + [pltpu.VMEM((CH, HH), f32)] * 4
        + [pltpu.SemaphoreType.DMA] * 9,
    )
    def body(h_a, h_b, dst, zrows, ns_a, ns_b,
             acc, ia0, ia1, ia2, ia3, ra0, ra1, ra2, ra3,
             li0, li1, li2, li3, lr0, lr1, lr2, lr3, sc):
        c = lax.axis_index("c")
        s = lax.axis_index("s")
        ebase = s * EPT
        ia, ra = (ia0, ia1, ia2, ia3), (ra0, ra1, ra2, ra3)
        li, lr = (li0, li1, li2, li3), (lr0, lr1, lr2, lr3)

        def core(h_h, ns_h):
            pltpu.sync_copy(zrows, acc.at[pl.ds(s * NPT, NPT)])
            plsc.subcore_barrier()
            _ph_scatter(h_h, dst, acc, ia, ra, li, lr, sc, ebase, EPT // CH)
            plsc.subcore_barrier()
            pltpu.sync_copy(acc.at[pl.ds(s * NPT, NPT)],
                            ns_h.at[pl.ds(s * NPT, NPT)])

        pl.when(c == 0)(lambda: core(h_a, ns_a))
        pl.when(c == 1)(lambda: core(h_b, ns_b))

    return body


@functools.cache
def _sc_pool():
    gpt = NG // NS

    nch = NPT // CH

    @functools.partial(
        pl.kernel, mesh=_mesh(),
        out_type=[jax.ShapeDtypeStruct((NG, HH), f32) for _ in range(2)],
        scratch_types=[pltpu.VMEM_SHARED((NG, HH), f32)]
        + [pltpu.VMEM((CH,), i32)] * 2 + [pltpu.VMEM((CH, HH), f32)] * 2
        + [pltpu.SemaphoreType.DMA] * 5,
    )
    def body(ne_a, ne_b, bat, zrows, p_a, p_b,
             acc, ia0, ia1, ra0, ra1, li0, li1, lr0, lr1, sc):
        c = lax.axis_index("c")
        s = lax.axis_index("s")
        nbase = s * NPT
        ia, ra, li, lr = (ia0, ia1), (ra0, ra1), (li0, li1), (lr0, lr1)

        def core(ne_h, p_h):
            dl = [None, None]
            dr = [None, None]
            for k in range(2):
                off = nbase + k * CH
                dl[k] = pltpu.async_copy(bat.at[pl.ds(off, CH)], ia[k], li[k])
                dr[k] = pltpu.async_copy(ne_h.at[pl.ds(off, CH)], ra[k],
                                         lr[k])
            pltpu.sync_copy(zrows.at[pl.ds(0, gpt)],
                            acc.at[pl.ds(s * gpt, gpt)])
            plsc.subcore_barrier()
            for k in range(nch):
                b = k % 2
                dl[b].wait()
                dr[b].wait()
                pltpu.async_copy(ra[b], acc.at[ia[b]], sc, add=True).wait()
                if k + 2 < nch:
                    off = nbase + (k + 2) * CH
                    dl[b] = pltpu.async_copy(bat.at[pl.ds(off, CH)], ia[b],
                                             li[b])
                    dr[b] = pltpu.async_copy(ne_h.at[pl.ds(off, CH)], ra[b],
                                             lr[b])
            plsc.subcore_barrier()
            pltpu.sync_copy(acc.at[pl.ds(s * gpt, gpt)],
                            p_h.at[pl.ds(s * gpt, gpt)])

        pl.when(c == 0)(lambda: core(ne_a, p_a))
        pl.when(c == 1)(lambda: core(ne_b, p_b))

    return body


BE = 2048


bf16 = jnp.bfloat16


def _tc_a_body(xg, ea, wi, wm, h0a, h0b, ma, mb):
    xe = xg[...] + jnp.pad(ea[...], ((0, 0), (6, 7)))
    h0 = _relu(jnp.dot(xe, wi[...], preferred_element_type=f32))
    m = _relu(jnp.dot(h0.astype(bf16), wm[...].astype(bf16),
                      preferred_element_type=f32))
    h0a[...] = h0[:, :HH].astype(bf16)
    h0b[...] = h0[:, HH:].astype(bf16)
    ma[...] = m[:, :HH]
    mb[...] = m[:, HH:]


def _tc_a(xg, ea, wi16, wm):
    eb = lambda i: (i, 0)
    return pl.pallas_call(
        _tc_a_body,
        grid=(EP // BE,),
        in_specs=[pl.BlockSpec((BE, 16), eb), pl.BlockSpec((BE, 3), eb),
                  pl.BlockSpec((16, H), lambda i: (0, 0)),
                  pl.BlockSpec((H, H), lambda i: (0, 0))],
        out_specs=[pl.BlockSpec((BE, HH), eb)] * 4,
        out_shape=[jax.ShapeDtypeStruct((EP, HH), bf16)] * 2
        + [jax.ShapeDtypeStruct((EP, HH), f32)] * 2,
    )(xg, ea, wi16, wm)


def _tc_b_body(h0a, h0b, g1a, g1b, g2a, g2b, wm, ma, mb):
    ha = h0a[...].astype(f32) + g1a[...] - g2a[...]
    hb = h0b[...].astype(f32) + g1b[...] - g2b[...]
    h = jnp.concatenate([ha, hb], axis=1)
    m = _relu(jnp.dot(h.astype(bf16), wm[...].astype(bf16),
                      preferred_element_type=f32))
    ma[...] = m[:, :HH]
    mb[...] = m[:, HH:]


def _tc_b(h0a, h0b, g1a, g1b, g2a, g2b, wm):
    eb = lambda i: (i, 0)
    return pl.pallas_call(
        _tc_b_body,
        grid=(EP // BE,),
        in_specs=[pl.BlockSpec((BE, HH), eb)] * 6
        + [pl.BlockSpec((H, H), lambda i: (0, 0))],
        out_specs=[pl.BlockSpec((BE, HH), eb)] * 2,
        out_shape=[jax.ShapeDtypeStruct((EP, HH), f32)] * 2,
    )(h0a, h0b, g1a, g1b, g2a, g2b, wm)


def _tc_c2_body(h0a, h0b, g1a, g1b, g2a, g2b, h3a, h3b):
    h3a[...] = h0a[...].astype(f32) + g1a[...] - g2a[...]
    h3b[...] = h0b[...].astype(f32) + g1b[...] - g2b[...]


def _tc_c2(h0a, h0b, g1a, g1b, g2a, g2b):
    eb = lambda i: (i, 0)
    return pl.pallas_call(
        _tc_c2_body,
        grid=(EP // BE,),
        in_specs=[pl.BlockSpec((BE, HH), eb)] * 6,
        out_specs=[pl.BlockSpec((BE, HH), eb)] * 2,
        out_shape=[jax.ShapeDtypeStruct((EP, HH), f32)] * 2,
    )(h0a, h0b, g1a, g1b, g2a, g2b)


def _tc_c_body(xp, nsa, nsb, wax, wah, nea, neb):
    ns = jnp.concatenate([nsa[...], nsb[...]], axis=1)
    ne = _relu(jnp.dot(xp[...], wax[...], preferred_element_type=f32)
               + jnp.dot(ns, wah[...], preferred_element_type=f32))
    nea[...] = ne[:, :HH]
    neb[...] = ne[:, HH:]


def _tc_c(x16, nsa, nsb, wax16, wah):
    nb = lambda i: (i, 0)
    nbk = 2048
    return pl.pallas_call(
        _tc_c_body,
        grid=(NP // nbk,),
        in_specs=[pl.BlockSpec((nbk, 16), nb), pl.BlockSpec((nbk, HH), nb),
                  pl.BlockSpec((nbk, HH), nb),
                  pl.BlockSpec((16, H), lambda i: (0, 0)),
                  pl.BlockSpec((H, H), lambda i: (0, 0))],
        out_specs=[pl.BlockSpec((nbk, HH), nb)] * 2,
        out_shape=[jax.ShapeDtypeStruct((NP, HH), f32)] * 2,
    )(x16, nsa, nsb, wax16, wah)


def _tc_d_body(pa, pb, w1, b1, w2, b2, wl, bl, out):
    p = jnp.concatenate([pa[...], pb[...]], axis=1)
    f1 = _relu(jnp.dot(p, w1[...], preferred_element_type=f32) + b1[...])
    f2 = jnp.dot(f1, w2[...], preferred_element_type=f32) + b2[...]
    out[...] = jnp.dot(f2, wl[...], preferred_element_type=f32) + bl[...]


def _tc_d(pa, pb, w1, b1, w2, b2, wl, bl):
    return pl.pallas_call(
        _tc_d_body,
        out_shape=jax.ShapeDtypeStruct((NG, 128), f32),
    )(pa, pb, w1, b1, w2, b2, wl, bl)


def kernel(x, edge_index, edge_attr, rev_edge, batch, depth,
           Wi, Wm, Wa, W1, b1, W2, b2, Wl, bl):
    src = edge_index[0].astype(i32)
    dst = edge_index[1].astype(i32)
    rev = rev_edge.astype(i32)
    bat = batch.astype(i32)

    padi = jnp.full((EP - NE,), NN, dtype=i32)
    src_p = jnp.concatenate([src, padi])
    dst_p = jnp.concatenate([dst, padi])
    rev_p = jnp.concatenate([rev, jnp.arange(NE, EP, dtype=i32)])
    ea_p = jnp.zeros((EP, 3), f32).at[:NE].set(edge_attr)
    x16 = jnp.zeros((NP, 16), f32).at[:NN, :6].set(x)
    bat_p = jnp.zeros((NP,), i32).at[:NN].set(bat)
    zrows = jnp.zeros((NPT, HH), f32)

    wi16 = jnp.zeros((16, H), f32).at[:9].set(Wi)
    wax16 = jnp.zeros((16, H), f32).at[:6].set(Wa[:6])
    wah = Wa[6:]

    (xg,) = _sc_gather_x()(x16, src_p)
    h0a, h0b, ma, mb = _tc_a(xg, ea_p, wi16, Wm)

    g1a = g1b = g2a = g2b = None
    for i in range(3):
        g1a, g1b, g2a, g2b = _sc_depth()(ma, mb, dst_p, src_p, rev_p, zrows)
        if i < 2:
            ma, mb = _tc_b(h0a, h0b, g1a, g1b, g2a, g2b, Wm)

    h3a, h3b = _tc_c2(h0a, h0b, g1a, g1b, g2a, g2b)
    nsa, nsb = _sc_segsum()(h3a, h3b, dst_p, zrows)
    nea, neb = _tc_c(x16, nsa, nsb, wax16, wah)
    pa, pb = _sc_pool()(nea, neb, bat_p, zrows)
    return _tc_d(pa, pb, W1, b1.reshape(1, -1), W2, b2.reshape(1, -1),
                 Wl, bl.reshape(1, -1))


# speedup vs baseline: 1.7320x; 1.0130x over previous
import functools

import jax
import jax.numpy as jnp
from jax import lax
from jax.experimental import pallas as pl
from jax.experimental.pallas import tpu as pltpu
from jax.experimental.pallas import tpu_sc as plsc

NN = 10000
NP = 10240
NE = 160000
EP = 163840
H = 256
HH = 128
NG = 256
NC, NS, CH = 2, 16, 64
EPT = EP // NS
EPW = EP // (NC * NS)
NPT = NP // NS

f32 = jnp.float32
i32 = jnp.int32


@functools.cache
def _mesh():
    return plsc.VectorSubcoreMesh(core_axis_name="c", subcore_axis_name="s",
                                  num_cores=NC, num_subcores=NS)


def _relu(v):
    return jnp.maximum(v, 0.0)


@functools.cache
def _sc_gather_x():
    @functools.partial(
        pl.kernel, mesh=_mesh(),
        out_type=[jax.ShapeDtypeStruct((EP, 16), f32)],
        scratch_types=[pltpu.VMEM((128,), i32), pltpu.VMEM((128,), i32),
                       pltpu.VMEM((128, 16), f32), pltpu.VMEM((128, 16), f32)]
        + [pltpu.SemaphoreType.DMA] * 6,
        compiler_params=pltpu.CompilerParams(use_tc_tiling_on_sc=False),
    )
    def body(x16, src, xg, idx0, idx1, row0, row1, li0, li1, q0, q1, w0, w1):
        c = lax.axis_index("c")
        s = lax.axis_index("s")
        ebase = (s * NC + c) * EPW
        idxb, rowb, li, q, w = (idx0, idx1), (row0, row1), (li0, li1), \
            (q0, q1), (w0, w1)
        CH = 128
        nch = EPW // CH

        for b in range(2):
            pltpu.async_copy(src.at[pl.ds(ebase + b * CH, CH)], idxb[b], li[b])

        def rnd(g, carry):
            dg = []
            for b in range(2):
                k = g * 2 + b
                off = pl.multiple_of(ebase + k * CH, CH)
                pltpu.make_async_copy(src.at[pl.ds(off, CH)], idxb[b],
                                      li[b]).wait()

                @pl.when(k >= 2)
                def _():
                    pltpu.make_async_copy(rowb[b], xg.at[pl.ds(off, CH)],
                                          w[b]).wait()

                dg.append(pltpu.async_copy(x16.at[idxb[b]], rowb[b], q[b]))
            for b in range(2):
                k = g * 2 + b
                off = pl.multiple_of(ebase + k * CH, CH)
                dg[b].wait()
                pltpu.async_copy(rowb[b], xg.at[pl.ds(off, CH)], w[b])

                @pl.when(k + 2 < nch)
                def _():
                    noff = pl.multiple_of(ebase + (k + 2) * CH, CH)
                    pltpu.async_copy(src.at[pl.ds(noff, CH)], idxb[b], li[b])
            return carry

        lax.fori_loop(0, nch // 2, rnd, 0)
        for b in range(2):
            pltpu.make_async_copy(rowb[b], xg.at[pl.ds(ebase, CH)],
                                  w[b]).wait()

    return body


def _ph_scatter(rows_src, idx_src, acc, ia, ra, li, lr, sc, ebase, nch):
    nb = len(ra)
    for b in range(nb):
        off = pl.multiple_of(ebase + b * CH, CH)
        pltpu.async_copy(idx_src.at[pl.ds(off, CH)], ia[b], li[b])
        pltpu.async_copy(rows_src.at[pl.ds(off, CH)], ra[b], lr[b])

    def rnd(g, carry):
        for half in range(2):
            dsc = []
            for j in range(2):
                b = half * 2 + j
                k = g * 4 + b
                off = pl.multiple_of(ebase + k * CH, CH)
                pltpu.make_async_copy(idx_src.at[pl.ds(off, CH)], ia[b],
                                      li[b]).wait()
                pltpu.make_async_copy(rows_src.at[pl.ds(off, CH)], ra[b],
                                      lr[b]).wait()
                dsc.append(pltpu.async_copy(ra[b], acc.at[ia[b]], sc,
                                            add=True))
            for j in range(2):
                b = half * 2 + j
                k = g * 4 + b
                dsc[j].wait()

                @pl.when(k + nb < nch)
                def _():
                    noff = pl.multiple_of(ebase + (k + nb) * CH, CH)
                    pltpu.async_copy(idx_src.at[pl.ds(noff, CH)], ia[b],
                                     li[b])
                    pltpu.async_copy(rows_src.at[pl.ds(noff, CH)], ra[b],
                                     lr[b])
        return carry

    lax.fori_loop(0, nch // 4, rnd, 0)


@functools.cache
def _sc_depth():
    @functools.partial(
        pl.kernel, mesh=_mesh(),
        out_type=[jax.ShapeDtypeStruct((EP, HH), f32) for _ in range(4)],
        scratch_types=[pltpu.VMEM_SHARED((NP, HH), f32)]
        + [pltpu.VMEM((CH,), i32)] * 4 + [pltpu.VMEM((CH, HH), f32)] * 4
        + [pltpu.SemaphoreType.DMA] * 17,
    )
    def body(m_a, m_b, dst, src, rev, zrows,
             g1_a, g1_b, g2_a, g2_b,
             acc, ia0, ia1, ia2, ia3, ra0, ra1, ra2, ra3,
             li0, li1, li2, li3, lr0, lr1, lr2, lr3, sc,
             q10, q11, q20, q21, w10, w11, w20, w21):
        c = lax.axis_index("c")
        s = lax.axis_index("s")
        ebase = s * EPT
        nch = EPT // CH
        ia = (ia0, ia1, ia2, ia3)
        ra = (ra0, ra1, ra2, ra3)
        li = (li0, li1, li2, li3)
        lr = (lr0, lr1, lr2, lr3)
        qs = (q10, q11, q20, q21)
        ws = (w10, w11, w20, w21)

        def core(m_h, g1_h, g2_h):
            pltpu.sync_copy(zrows, acc.at[pl.ds(s * NPT, NPT)])
            plsc.subcore_barrier()
            _ph_scatter(m_h, dst, acc, ia, ra, li, lr, sc, ebase, nch)
            for b in range(2):
                off = pl.multiple_of(ebase + b * CH, CH)
                pltpu.async_copy(src.at[pl.ds(off, CH)], ia[b], li[b])
                pltpu.async_copy(rev.at[pl.ds(off, CH)], ia[2 + b], li[2 + b])
            plsc.subcore_barrier()

            def ph_c(g, carry):
                dsc = []
                for b in range(2):
                    k = g * 2 + b
                    off = pl.multiple_of(ebase + k * CH, CH)
                    pltpu.make_async_copy(src.at[pl.ds(off, CH)], ia[b],
                                          li[b]).wait()
                    pltpu.make_async_copy(rev.at[pl.ds(off, CH)], ia[2 + b],
                                          li[2 + b]).wait()

                    @pl.when(k >= 2)
                    def _():
                        pltpu.make_async_copy(ra[b], g1_h.at[pl.ds(off, CH)],
                                              ws[b]).wait()
                        pltpu.make_async_copy(ra[2 + b],
                                              g2_h.at[pl.ds(off, CH)],
                                              ws[2 + b]).wait()

                    d2 = pltpu.async_copy(m_h.at[ia[2 + b]], ra[2 + b],
                                          qs[2 + b])
                    d1 = pltpu.async_copy(acc.at[ia[b]], ra[b], qs[b])
                    dsc.append((d1, d2))
                for b in range(2):
                    k = g * 2 + b
                    off = pl.multiple_of(ebase + k * CH, CH)
                    d1, d2 = dsc[b]
                    d1.wait()
                    d2.wait()
                    pltpu.async_copy(ra[b], g1_h.at[pl.ds(off, CH)], ws[b])
                    pltpu.async_copy(ra[2 + b], g2_h.at[pl.ds(off, CH)],
                                     ws[2 + b])

                    @pl.when(k + 2 < nch)
                    def _():
                        noff = pl.multiple_of(ebase + (k + 2) * CH, CH)
                        pltpu.async_copy(src.at[pl.ds(noff, CH)], ia[b],
                                         li[b])
                        pltpu.async_copy(rev.at[pl.ds(noff, CH)], ia[2 + b],
                                         li[2 + b])
                return carry

            lax.fori_loop(0, nch // 2, ph_c, 0)
            for b in range(2):
                pltpu.make_async_copy(ra[b], g1_h.at[pl.ds(ebase, CH)],
                                      ws[b]).wait()
                pltpu.make_async_copy(ra[2 + b], g2_h.at[pl.ds(ebase, CH)],
                                      ws[2 + b]).wait()

        pl.when(c == 0)(lambda: core(m_a, g1_a, g2_a))
        pl.when(c == 1)(lambda: core(m_b, g1_b, g2_b))

    return body


@functools.cache
def _sc_segsum():
    @functools.partial(
        pl.kernel, mesh=_mesh(),
        out_type=[jax.ShapeDtypeStruct((NP, HH), f32) for _ in range(2)],
        scratch_types=[pltpu.VMEM_SHARED((NP, HH), f32)]
        + [pltpu.VMEM((CH,), i32)] * 4 + [pltpu.VMEM((CH, HH), f32)] * 4
        + [pltpu.SemaphoreType.DMA] * 9,
    )
    def body(h_a, h_b, dst, zrows, ns_a, ns_b,
             acc, ia0, ia1, ia2, ia3, ra0, ra1, ra2, ra3,
             li0, li1, li2, li3, lr0, lr1, lr2, lr3, sc):
        c = lax.axis_index("c")
        s = lax.axis_index("s")
        ebase = s * EPT
        ia, ra = (ia0, ia1, ia2, ia3), (ra0, ra1, ra2, ra3)
        li, lr = (li0, li1, li2, li3), (lr0, lr1, lr2, lr3)

        def core(h_h, ns_h):
            pltpu.sync_copy(zrows, acc.at[pl.ds(s * NPT, NPT)])
            plsc.subcore_barrier()
            _ph_scatter(h_h, dst, acc, ia, ra, li, lr, sc, ebase, EPT // CH)
            plsc.subcore_barrier()
            pltpu.sync_copy(acc.at[pl.ds(s * NPT, NPT)],
                            ns_h.at[pl.ds(s * NPT, NPT)])

        pl.when(c == 0)(lambda: core(h_a, ns_a))
        pl.when(c == 1)(lambda: core(h_b, ns_b))

    return body


@functools.cache
def _sc_pool():
    gpt = NG // NS

    nch = NPT // CH

    @functools.partial(
        pl.kernel, mesh=_mesh(),
        out_type=[jax.ShapeDtypeStruct((NG, HH), f32) for _ in range(2)],
        scratch_types=[pltpu.VMEM_SHARED((NG, HH), f32)]
        + [pltpu.VMEM((CH,), i32)] * 2 + [pltpu.VMEM((CH, HH), f32)] * 2
        + [pltpu.SemaphoreType.DMA] * 5,
    )
    def body(ne_a, ne_b, bat, zrows, p_a, p_b,
             acc, ia0, ia1, ra0, ra1, li0, li1, lr0, lr1, sc):
        c = lax.axis_index("c")
        s = lax.axis_index("s")
        nbase = s * NPT
        ia, ra, li, lr = (ia0, ia1), (ra0, ra1), (li0, li1), (lr0, lr1)

        def core(ne_h, p_h):
            dl = [None, None]
            dr = [None, None]
            for k in range(2):
                off = nbase + k * CH
                dl[k] = pltpu.async_copy(bat.at[pl.ds(off, CH)], ia[k], li[k])
                dr[k] = pltpu.async_copy(ne_h.at[pl.ds(off, CH)], ra[k],
                                         lr[k])
            pltpu.sync_copy(zrows.at[pl.ds(0, gpt)],
                            acc.at[pl.ds(s * gpt, gpt)])
            plsc.subcore_barrier()
            for k in range(nch):
                b = k % 2
                dl[b].wait()
                dr[b].wait()
                pltpu.async_copy(ra[b], acc.at[ia[b]], sc, add=True).wait()
                if k + 2 < nch:
                    off = nbase + (k + 2) * CH
                    dl[b] = pltpu.async_copy(bat.at[pl.ds(off, CH)], ia[b],
                                             li[b])
                    dr[b] = pltpu.async_copy(ne_h.at[pl.ds(off, CH)], ra[b],
                                             lr[b])
            plsc.subcore_barrier()
            pltpu.sync_copy(acc.at[pl.ds(s * gpt, gpt)],
                            p_h.at[pl.ds(s * gpt, gpt)])

        pl.when(c == 0)(lambda: core(ne_a, p_a))
        pl.when(c == 1)(lambda: core(ne_b, p_b))

    return body


BE = 4096


bf16 = jnp.bfloat16


def _tc_a_body(xg, ea, wi, wm, h0a, h0b, ma, mb):
    xe = xg[...] + jnp.pad(ea[...], ((0, 0), (6, 7)))
    h0 = _relu(jnp.dot(xe, wi[...], preferred_element_type=f32))
    m = _relu(jnp.dot(h0.astype(bf16), wm[...].astype(bf16),
                      preferred_element_type=f32))
    h0a[...] = h0[:, :HH].astype(bf16)
    h0b[...] = h0[:, HH:].astype(bf16)
    ma[...] = m[:, :HH]
    mb[...] = m[:, HH:]


def _tc_a(xg, ea, wi16, wm):
    eb = lambda i: (i, 0)
    return pl.pallas_call(
        _tc_a_body,
        grid=(EP // BE,),
        in_specs=[pl.BlockSpec((BE, 16), eb), pl.BlockSpec((BE, 3), eb),
                  pl.BlockSpec((16, H), lambda i: (0, 0)),
                  pl.BlockSpec((H, H), lambda i: (0, 0))],
        out_specs=[pl.BlockSpec((BE, HH), eb)] * 4,
        out_shape=[jax.ShapeDtypeStruct((EP, HH), bf16)] * 2
        + [jax.ShapeDtypeStruct((EP, HH), f32)] * 2,
    )(xg, ea, wi16, wm)


def _tc_b_body(h0a, h0b, g1a, g1b, g2a, g2b, wm, ma, mb):
    ha = h0a[...].astype(f32) + g1a[...] - g2a[...]
    hb = h0b[...].astype(f32) + g1b[...] - g2b[...]
    h = jnp.concatenate([ha, hb], axis=1)
    m = _relu(jnp.dot(h.astype(bf16), wm[...].astype(bf16),
                      preferred_element_type=f32))
    ma[...] = m[:, :HH]
    mb[...] = m[:, HH:]


def _tc_b(h0a, h0b, g1a, g1b, g2a, g2b, wm):
    eb = lambda i: (i, 0)
    return pl.pallas_call(
        _tc_b_body,
        grid=(EP // BE,),
        in_specs=[pl.BlockSpec((BE, HH), eb)] * 6
        + [pl.BlockSpec((H, H), lambda i: (0, 0))],
        out_specs=[pl.BlockSpec((BE, HH), eb)] * 2,
        out_shape=[jax.ShapeDtypeStruct((EP, HH), f32)] * 2,
    )(h0a, h0b, g1a, g1b, g2a, g2b, wm)


def _tc_c2_body(h0a, h0b, g1a, g1b, g2a, g2b, h3a, h3b):
    h3a[...] = h0a[...].astype(f32) + g1a[...] - g2a[...]
    h3b[...] = h0b[...].astype(f32) + g1b[...] - g2b[...]


def _tc_c2(h0a, h0b, g1a, g1b, g2a, g2b):
    eb = lambda i: (i, 0)
    return pl.pallas_call(
        _tc_c2_body,
        grid=(EP // BE,),
        in_specs=[pl.BlockSpec((BE, HH), eb)] * 6,
        out_specs=[pl.BlockSpec((BE, HH), eb)] * 2,
        out_shape=[jax.ShapeDtypeStruct((EP, HH), f32)] * 2,
    )(h0a, h0b, g1a, g1b, g2a, g2b)


def _tc_c_body(xp, nsa, nsb, wax, wah, nea, neb):
    ns = jnp.concatenate([nsa[...], nsb[...]], axis=1)
    ne = _relu(jnp.dot(xp[...], wax[...], preferred_element_type=f32)
               + jnp.dot(ns, wah[...], preferred_element_type=f32))
    nea[...] = ne[:, :HH]
    neb[...] = ne[:, HH:]


def _tc_c(x16, nsa, nsb, wax16, wah):
    nb = lambda i: (i, 0)
    nbk = 2048
    return pl.pallas_call(
        _tc_c_body,
        grid=(NP // nbk,),
        in_specs=[pl.BlockSpec((nbk, 16), nb), pl.BlockSpec((nbk, HH), nb),
                  pl.BlockSpec((nbk, HH), nb),
                  pl.BlockSpec((16, H), lambda i: (0, 0)),
                  pl.BlockSpec((H, H), lambda i: (0, 0))],
        out_specs=[pl.BlockSpec((nbk, HH), nb)] * 2,
        out_shape=[jax.ShapeDtypeStruct((NP, HH), f32)] * 2,
    )(x16, nsa, nsb, wax16, wah)


def _tc_d_body(pa, pb, w1, b1, w2, b2, wl, bl, out):
    p = jnp.concatenate([pa[...], pb[...]], axis=1)
    f1 = _relu(jnp.dot(p, w1[...], preferred_element_type=f32) + b1[...])
    f2 = jnp.dot(f1, w2[...], preferred_element_type=f32) + b2[...]
    out[...] = jnp.dot(f2, wl[...], preferred_element_type=f32) + bl[...]


def _tc_d(pa, pb, w1, b1, w2, b2, wl, bl):
    return pl.pallas_call(
        _tc_d_body,
        out_shape=jax.ShapeDtypeStruct((NG, 128), f32),
    )(pa, pb, w1, b1, w2, b2, wl, bl)


def kernel(x, edge_index, edge_attr, rev_edge, batch, depth,
           Wi, Wm, Wa, W1, b1, W2, b2, Wl, bl):
    src = edge_index[0].astype(i32)
    dst = edge_index[1].astype(i32)
    rev = rev_edge.astype(i32)
    bat = batch.astype(i32)

    padi = jnp.full((EP - NE,), NN, dtype=i32)
    src_p = jnp.concatenate([src, padi])
    dst_p = jnp.concatenate([dst, padi])
    rev_p = jnp.concatenate([rev, jnp.arange(NE, EP, dtype=i32)])
    ea_p = jnp.zeros((EP, 3), f32).at[:NE].set(edge_attr)
    x16 = jnp.zeros((NP, 16), f32).at[:NN, :6].set(x)
    bat_p = jnp.zeros((NP,), i32).at[:NN].set(bat)
    zrows = jnp.zeros((NPT, HH), f32)

    wi16 = jnp.zeros((16, H), f32).at[:9].set(Wi)
    wax16 = jnp.zeros((16, H), f32).at[:6].set(Wa[:6])
    wah = Wa[6:]

    (xg,) = _sc_gather_x()(x16, src_p)
    h0a, h0b, ma, mb = _tc_a(xg, ea_p, wi16, Wm)

    g1a = g1b = g2a = g2b = None
    for i in range(3):
        g1a, g1b, g2a, g2b = _sc_depth()(ma, mb, dst_p, src_p, rev_p, zrows)
        if i < 2:
            ma, mb = _tc_b(h0a, h0b, g1a, g1b, g2a, g2b, Wm)

    h3a, h3b = _tc_c2(h0a, h0b, g1a, g1b, g2a, g2b)
    nsa, nsb = _sc_segsum()(h3a, h3b, dst_p, zrows)
    nea, neb = _tc_c(x16, nsa, nsb, wax16, wah)
    pa, pb = _sc_pool()(nea, neb, bat_p, zrows)
    return _tc_d(pa, pb, W1, b1.reshape(1, -1), W2, b2.reshape(1, -1),
                 Wl, bl.reshape(1, -1))


# speedup vs baseline: 1.7335x; 1.0009x over previous
import functools

import jax
import jax.numpy as jnp
from jax import lax
from jax.experimental import pallas as pl
from jax.experimental.pallas import tpu as pltpu
from jax.experimental.pallas import tpu_sc as plsc

NN = 10000
NP = 10240
NE = 160000
EP = 163840
H = 256
HH = 128
NG = 256
NC, NS, CH = 2, 16, 64
EPT = EP // NS
EPW = EP // (NC * NS)
NPT = NP // NS

f32 = jnp.float32
i32 = jnp.int32


@functools.cache
def _mesh():
    return plsc.VectorSubcoreMesh(core_axis_name="c", subcore_axis_name="s",
                                  num_cores=NC, num_subcores=NS)


def _relu(v):
    return jnp.maximum(v, 0.0)


@functools.cache
def _sc_gather_x():
    @functools.partial(
        pl.kernel, mesh=_mesh(),
        out_type=[jax.ShapeDtypeStruct((EP, 16), f32)],
        scratch_types=[pltpu.VMEM((128,), i32), pltpu.VMEM((128,), i32),
                       pltpu.VMEM((128, 16), f32), pltpu.VMEM((128, 16), f32)]
        + [pltpu.SemaphoreType.DMA] * 6,
        compiler_params=pltpu.CompilerParams(use_tc_tiling_on_sc=False),
    )
    def body(x16, src, xg, idx0, idx1, row0, row1, li0, li1, q0, q1, w0, w1):
        c = lax.axis_index("c")
        s = lax.axis_index("s")
        ebase = (s * NC + c) * EPW
        idxb, rowb, li, q, w = (idx0, idx1), (row0, row1), (li0, li1), \
            (q0, q1), (w0, w1)
        CH = 128
        nch = EPW // CH

        for b in range(2):
            pltpu.async_copy(src.at[pl.ds(ebase + b * CH, CH)], idxb[b], li[b])

        def rnd(g, carry):
            dg = []
            for b in range(2):
                k = g * 2 + b
                off = pl.multiple_of(ebase + k * CH, CH)
                pltpu.make_async_copy(src.at[pl.ds(off, CH)], idxb[b],
                                      li[b]).wait()

                @pl.when(k >= 2)
                def _():
                    pltpu.make_async_copy(rowb[b], xg.at[pl.ds(off, CH)],
                                          w[b]).wait()

                dg.append(pltpu.async_copy(x16.at[idxb[b]], rowb[b], q[b]))
            for b in range(2):
                k = g * 2 + b
                off = pl.multiple_of(ebase + k * CH, CH)
                dg[b].wait()
                pltpu.async_copy(rowb[b], xg.at[pl.ds(off, CH)], w[b])

                @pl.when(k + 2 < nch)
                def _():
                    noff = pl.multiple_of(ebase + (k + 2) * CH, CH)
                    pltpu.async_copy(src.at[pl.ds(noff, CH)], idxb[b], li[b])
            return carry

        lax.fori_loop(0, nch // 2, rnd, 0)
        for b in range(2):
            pltpu.make_async_copy(rowb[b], xg.at[pl.ds(ebase, CH)],
                                  w[b]).wait()

    return body


def _ph_scatter(rows_src, idx_src, acc, ia, ra, li, lr, sc, ebase, nch):
    nb = len(ra)
    for b in range(nb):
        off = pl.multiple_of(ebase + b * CH, CH)
        pltpu.async_copy(idx_src.at[pl.ds(off, CH)], ia[b], li[b])
        pltpu.async_copy(rows_src.at[pl.ds(off, CH)], ra[b], lr[b])

    def rnd(g, carry):
        for half in range(2):
            dsc = []
            for j in range(2):
                b = half * 2 + j
                k = g * 4 + b
                off = pl.multiple_of(ebase + k * CH, CH)
                pltpu.make_async_copy(idx_src.at[pl.ds(off, CH)], ia[b],
                                      li[b]).wait()
                pltpu.make_async_copy(rows_src.at[pl.ds(off, CH)], ra[b],
                                      lr[b]).wait()
                dsc.append(pltpu.async_copy(ra[b], acc.at[ia[b]], sc,
                                            add=True))
            for j in range(2):
                b = half * 2 + j
                k = g * 4 + b
                dsc[j].wait()

                @pl.when(k + nb < nch)
                def _():
                    noff = pl.multiple_of(ebase + (k + nb) * CH, CH)
                    pltpu.async_copy(idx_src.at[pl.ds(noff, CH)], ia[b],
                                     li[b])
                    pltpu.async_copy(rows_src.at[pl.ds(noff, CH)], ra[b],
                                     lr[b])
        return carry

    lax.fori_loop(0, nch // 4, rnd, 0)


@functools.cache
def _sc_depth():
    @functools.partial(
        pl.kernel, mesh=_mesh(),
        out_type=[jax.ShapeDtypeStruct((EP, HH), f32) for _ in range(4)],
        scratch_types=[pltpu.VMEM_SHARED((NP, HH), f32)]
        + [pltpu.VMEM((CH,), i32)] * 4 + [pltpu.VMEM((CH, HH), f32)] * 4
        + [pltpu.SemaphoreType.DMA] * 17,
    )
    def body(m_a, m_b, dst, src, rev, zrows,
             g1_a, g1_b, g2_a, g2_b,
             acc, ia0, ia1, ia2, ia3, ra0, ra1, ra2, ra3,
             li0, li1, li2, li3, lr0, lr1, lr2, lr3, sc,
             q10, q11, q20, q21, w10, w11, w20, w21):
        c = lax.axis_index("c")
        s = lax.axis_index("s")
        ebase = s * EPT
        nch = EPT // CH
        ia = (ia0, ia1, ia2, ia3)
        ra = (ra0, ra1, ra2, ra3)
        li = (li0, li1, li2, li3)
        lr = (lr0, lr1, lr2, lr3)
        qs = (q10, q11, q20, q21)
        ws = (w10, w11, w20, w21)

        def core(m_h, g1_h, g2_h):
            pltpu.sync_copy(zrows, acc.at[pl.ds(s * NPT, NPT)])
            plsc.subcore_barrier()
            _ph_scatter(m_h, dst, acc, ia, ra, li, lr, sc, ebase, nch)
            for b in range(2):
                off = pl.multiple_of(ebase + b * CH, CH)
                pltpu.async_copy(src.at[pl.ds(off, CH)], ia[b], li[b])
                pltpu.async_copy(rev.at[pl.ds(off, CH)], ia[2 + b], li[2 + b])
            plsc.subcore_barrier()

            def ph_c(g, carry):
                dsc = []
                for b in range(2):
                    k = g * 2 + b
                    off = pl.multiple_of(ebase + k * CH, CH)
                    pltpu.make_async_copy(src.at[pl.ds(off, CH)], ia[b],
                                          li[b]).wait()
                    pltpu.make_async_copy(rev.at[pl.ds(off, CH)], ia[2 + b],
                                          li[2 + b]).wait()

                    @pl.when(k >= 2)
                    def _():
                        pltpu.make_async_copy(ra[b], g1_h.at[pl.ds(off, CH)],
                                              ws[b]).wait()
                        pltpu.make_async_copy(ra[2 + b],
                                              g2_h.at[pl.ds(off, CH)],
                                              ws[2 + b]).wait()

                    d2 = pltpu.async_copy(m_h.at[ia[2 + b]], ra[2 + b],
                                          qs[2 + b])
                    d1 = pltpu.async_copy(acc.at[ia[b]], ra[b], qs[b])
                    dsc.append((d1, d2))
                for b in range(2):
                    k = g * 2 + b
                    off = pl.multiple_of(ebase + k * CH, CH)
                    d1, d2 = dsc[b]
                    d1.wait()
                    d2.wait()
                    pltpu.async_copy(ra[b], g1_h.at[pl.ds(off, CH)], ws[b])
                    pltpu.async_copy(ra[2 + b], g2_h.at[pl.ds(off, CH)],
                                     ws[2 + b])

                    @pl.when(k + 2 < nch)
                    def _():
                        noff = pl.multiple_of(ebase + (k + 2) * CH, CH)
                        pltpu.async_copy(src.at[pl.ds(noff, CH)], ia[b],
                                         li[b])
                        pltpu.async_copy(rev.at[pl.ds(noff, CH)], ia[2 + b],
                                         li[2 + b])
                return carry

            lax.fori_loop(0, nch // 2, ph_c, 0)
            for b in range(2):
                pltpu.make_async_copy(ra[b], g1_h.at[pl.ds(ebase, CH)],
                                      ws[b]).wait()
                pltpu.make_async_copy(ra[2 + b], g2_h.at[pl.ds(ebase, CH)],
                                      ws[2 + b]).wait()

        pl.when(c == 0)(lambda: core(m_a, g1_a, g2_a))
        pl.when(c == 1)(lambda: core(m_b, g1_b, g2_b))

    return body


@functools.cache
def _sc_segsum():
    @functools.partial(
        pl.kernel, mesh=_mesh(),
        out_type=[jax.ShapeDtypeStruct((NP, HH), f32) for _ in range(2)],
        scratch_types=[pltpu.VMEM_SHARED((NP, HH), f32)]
        + [pltpu.VMEM((CH,), i32)] * 4 + [pltpu.VMEM((CH, HH), f32)] * 4
        + [pltpu.SemaphoreType.DMA] * 9,
    )
    def body(h_a, h_b, dst, zrows, ns_a, ns_b,
             acc, ia0, ia1, ia2, ia3, ra0, ra1, ra2, ra3,
             li0, li1, li2, li3, lr0, lr1, lr2, lr3, sc):
        c = lax.axis_index("c")
        s = lax.axis_index("s")
        ebase = s * EPT
        ia, ra = (ia0, ia1, ia2, ia3), (ra0, ra1, ra2, ra3)
        li, lr = (li0, li1, li2, li3), (lr0, lr1, lr2, lr3)

        def core(h_h, ns_h):
            pltpu.sync_copy(zrows, acc.at[pl.ds(s * NPT, NPT)])
            plsc.subcore_barrier()
            _ph_scatter(h_h, dst, acc, ia, ra, li, lr, sc, ebase, EPT // CH)
            plsc.subcore_barrier()
            pltpu.sync_copy(acc.at[pl.ds(s * NPT, NPT)],
                            ns_h.at[pl.ds(s * NPT, NPT)])

        pl.when(c == 0)(lambda: core(h_a, ns_a))
        pl.when(c == 1)(lambda: core(h_b, ns_b))

    return body


@functools.cache
def _sc_pool():
    gpt = NG // NS

    nch = NPT // CH

    @functools.partial(
        pl.kernel, mesh=_mesh(),
        out_type=[jax.ShapeDtypeStruct((NG, HH), f32) for _ in range(2)],
        scratch_types=[pltpu.VMEM_SHARED((NG, HH), f32)]
        + [pltpu.VMEM((CH,), i32)] * 2 + [pltpu.VMEM((CH, HH), f32)] * 2
        + [pltpu.SemaphoreType.DMA] * 5,
    )
    def body(ne_a, ne_b, bat, zrows, p_a, p_b,
             acc, ia0, ia1, ra0, ra1, li0, li1, lr0, lr1, sc):
        c = lax.axis_index("c")
        s = lax.axis_index("s")
        nbase = s * NPT
        ia, ra, li, lr = (ia0, ia1), (ra0, ra1), (li0, li1), (lr0, lr1)

        def core(ne_h, p_h):
            dl = [None, None]
            dr = [None, None]
            for k in range(2):
                off = nbase + k * CH
                dl[k] = pltpu.async_copy(bat.at[pl.ds(off, CH)], ia[k], li[k])
                dr[k] = pltpu.async_copy(ne_h.at[pl.ds(off, CH)], ra[k],
                                         lr[k])
            pltpu.sync_copy(zrows.at[pl.ds(0, gpt)],
                            acc.at[pl.ds(s * gpt, gpt)])
            plsc.subcore_barrier()
            for k in range(nch):
                b = k % 2
                dl[b].wait()
                dr[b].wait()
                pltpu.async_copy(ra[b], acc.at[ia[b]], sc, add=True).wait()
                if k + 2 < nch:
                    off = nbase + (k + 2) * CH
                    dl[b] = pltpu.async_copy(bat.at[pl.ds(off, CH)], ia[b],
                                             li[b])
                    dr[b] = pltpu.async_copy(ne_h.at[pl.ds(off, CH)], ra[b],
                                             lr[b])
            plsc.subcore_barrier()
            pltpu.sync_copy(acc.at[pl.ds(s * gpt, gpt)],
                            p_h.at[pl.ds(s * gpt, gpt)])

        pl.when(c == 0)(lambda: core(ne_a, p_a))
        pl.when(c == 1)(lambda: core(ne_b, p_b))

    return body


BE = 8192


bf16 = jnp.bfloat16


def _tc_a_body(xg, ea, wi, wm, h0a, h0b, ma, mb):
    xe = xg[...] + jnp.pad(ea[...], ((0, 0), (6, 7)))
    h0 = _relu(jnp.dot(xe, wi[...], preferred_element_type=f32))
    m = _relu(jnp.dot(h0.astype(bf16), wm[...].astype(bf16),
                      preferred_element_type=f32))
    h0a[...] = h0[:, :HH].astype(bf16)
    h0b[...] = h0[:, HH:].astype(bf16)
    ma[...] = m[:, :HH]
    mb[...] = m[:, HH:]


def _tc_a(xg, ea, wi16, wm):
    eb = lambda i: (i, 0)
    return pl.pallas_call(
        _tc_a_body,
        grid=(EP // BE,),
        in_specs=[pl.BlockSpec((BE, 16), eb), pl.BlockSpec((BE, 3), eb),
                  pl.BlockSpec((16, H), lambda i: (0, 0)),
                  pl.BlockSpec((H, H), lambda i: (0, 0))],
        out_specs=[pl.BlockSpec((BE, HH), eb)] * 4,
        out_shape=[jax.ShapeDtypeStruct((EP, HH), bf16)] * 2
        + [jax.ShapeDtypeStruct((EP, HH), f32)] * 2,
    )(xg, ea, wi16, wm)


def _tc_b_body(h0a, h0b, g1a, g1b, g2a, g2b, wm, ma, mb):
    ha = h0a[...].astype(f32) + g1a[...] - g2a[...]
    hb = h0b[...].astype(f32) + g1b[...] - g2b[...]
    h = jnp.concatenate([ha, hb], axis=1)
    m = _relu(jnp.dot(h.astype(bf16), wm[...].astype(bf16),
                      preferred_element_type=f32))
    ma[...] = m[:, :HH]
    mb[...] = m[:, HH:]


def _tc_b(h0a, h0b, g1a, g1b, g2a, g2b, wm):
    eb = lambda i: (i, 0)
    return pl.pallas_call(
        _tc_b_body,
        grid=(EP // BE,),
        in_specs=[pl.BlockSpec((BE, HH), eb)] * 6
        + [pl.BlockSpec((H, H), lambda i: (0, 0))],
        out_specs=[pl.BlockSpec((BE, HH), eb)] * 2,
        out_shape=[jax.ShapeDtypeStruct((EP, HH), f32)] * 2,
    )(h0a, h0b, g1a, g1b, g2a, g2b, wm)


def _tc_c2_body(h0a, h0b, g1a, g1b, g2a, g2b, h3a, h3b):
    h3a[...] = h0a[...].astype(f32) + g1a[...] - g2a[...]
    h3b[...] = h0b[...].astype(f32) + g1b[...] - g2b[...]


def _tc_c2(h0a, h0b, g1a, g1b, g2a, g2b):
    eb = lambda i: (i, 0)
    return pl.pallas_call(
        _tc_c2_body,
        grid=(EP // BE,),
        in_specs=[pl.BlockSpec((BE, HH), eb)] * 6,
        out_specs=[pl.BlockSpec((BE, HH), eb)] * 2,
        out_shape=[jax.ShapeDtypeStruct((EP, HH), f32)] * 2,
    )(h0a, h0b, g1a, g1b, g2a, g2b)


def _tc_c_body(xp, nsa, nsb, wax, wah, nea, neb):
    ns = jnp.concatenate([nsa[...], nsb[...]], axis=1)
    ne = _relu(jnp.dot(xp[...], wax[...], preferred_element_type=f32)
               + jnp.dot(ns, wah[...], preferred_element_type=f32))
    nea[...] = ne[:, :HH]
    neb[...] = ne[:, HH:]


def _tc_c(x16, nsa, nsb, wax16, wah):
    nb = lambda i: (i, 0)
    nbk = 2048
    return pl.pallas_call(
        _tc_c_body,
        grid=(NP // nbk,),
        in_specs=[pl.BlockSpec((nbk, 16), nb), pl.BlockSpec((nbk, HH), nb),
                  pl.BlockSpec((nbk, HH), nb),
                  pl.BlockSpec((16, H), lambda i: (0, 0)),
                  pl.BlockSpec((H, H), lambda i: (0, 0))],
        out_specs=[pl.BlockSpec((nbk, HH), nb)] * 2,
        out_shape=[jax.ShapeDtypeStruct((NP, HH), f32)] * 2,
    )(x16, nsa, nsb, wax16, wah)


def _tc_d_body(pa, pb, w1, b1, w2, b2, wl, bl, out):
    p = jnp.concatenate([pa[...], pb[...]], axis=1)
    f1 = _relu(jnp.dot(p, w1[...], preferred_element_type=f32) + b1[...])
    f2 = jnp.dot(f1, w2[...], preferred_element_type=f32) + b2[...]
    out[...] = jnp.dot(f2, wl[...], preferred_element_type=f32) + bl[...]


def _tc_d(pa, pb, w1, b1, w2, b2, wl, bl):
    return pl.pallas_call(
        _tc_d_body,
        out_shape=jax.ShapeDtypeStruct((NG, 128), f32),
    )(pa, pb, w1, b1, w2, b2, wl, bl)


def kernel(x, edge_index, edge_attr, rev_edge, batch, depth,
           Wi, Wm, Wa, W1, b1, W2, b2, Wl, bl):
    src = edge_index[0].astype(i32)
    dst = edge_index[1].astype(i32)
    rev = rev_edge.astype(i32)
    bat = batch.astype(i32)

    padi = jnp.full((EP - NE,), NN, dtype=i32)
    src_p = jnp.concatenate([src, padi])
    dst_p = jnp.concatenate([dst, padi])
    rev_p = jnp.concatenate([rev, jnp.arange(NE, EP, dtype=i32)])
    ea_p = jnp.zeros((EP, 3), f32).at[:NE].set(edge_attr)
    x16 = jnp.zeros((NP, 16), f32).at[:NN, :6].set(x)
    bat_p = jnp.zeros((NP,), i32).at[:NN].set(bat)
    zrows = jnp.zeros((NPT, HH), f32)

    wi16 = jnp.zeros((16, H), f32).at[:9].set(Wi)
    wax16 = jnp.zeros((16, H), f32).at[:6].set(Wa[:6])
    wah = Wa[6:]

    (xg,) = _sc_gather_x()(x16, src_p)
    h0a, h0b, ma, mb = _tc_a(xg, ea_p, wi16, Wm)

    g1a = g1b = g2a = g2b = None
    for i in range(3):
        g1a, g1b, g2a, g2b = _sc_depth()(ma, mb, dst_p, src_p, rev_p, zrows)
        if i < 2:
            ma, mb = _tc_b(h0a, h0b, g1a, g1b, g2a, g2b, Wm)

    h3a, h3b = _tc_c2(h0a, h0b, g1a, g1b, g2a, g2b)
    nsa, nsb = _sc_segsum()(h3a, h3b, dst_p, zrows)
    nea, neb = _tc_c(x16, nsa, nsb, wax16, wah)
    pa, pb = _sc_pool()(nea, neb, bat_p, zrows)
    return _tc_d(pa, pb, W1, b1.reshape(1, -1), W2, b2.reshape(1, -1),
                 Wl, bl.reshape(1, -1))


# speedup vs baseline: 1.7340x; 1.0002x over previous
import functools

import jax
import jax.numpy as jnp
from jax import lax
from jax.experimental import pallas as pl
from jax.experimental.pallas import tpu as pltpu
from jax.experimental.pallas import tpu_sc as plsc

NN = 10000
NP = 10240
NE = 160000
EP = 163840
H = 256
HH = 128
NG = 256
NC, NS, CH = 2, 16, 64
EPT = EP // NS
EPW = EP // (NC * NS)
NPT = NP // NS

f32 = jnp.float32
i32 = jnp.int32


@functools.cache
def _mesh():
    return plsc.VectorSubcoreMesh(core_axis_name="c", subcore_axis_name="s",
                                  num_cores=NC, num_subcores=NS)


def _relu(v):
    return jnp.maximum(v, 0.0)


@functools.cache
def _sc_gather_x():
    @functools.partial(
        pl.kernel, mesh=_mesh(),
        out_type=[jax.ShapeDtypeStruct((EP, 16), f32)],
        scratch_types=[pltpu.VMEM((128,), i32), pltpu.VMEM((128,), i32),
                       pltpu.VMEM((128, 16), f32), pltpu.VMEM((128, 16), f32)]
        + [pltpu.SemaphoreType.DMA] * 6,
        compiler_params=pltpu.CompilerParams(use_tc_tiling_on_sc=False),
    )
    def body(x16, src, xg, idx0, idx1, row0, row1, li0, li1, q0, q1, w0, w1):
        c = lax.axis_index("c")
        s = lax.axis_index("s")
        ebase = (s * NC + c) * EPW
        idxb, rowb, li, q, w = (idx0, idx1), (row0, row1), (li0, li1), \
            (q0, q1), (w0, w1)
        CH = 128
        nch = EPW // CH

        for b in range(2):
            pltpu.async_copy(src.at[pl.ds(ebase + b * CH, CH)], idxb[b], li[b])

        def rnd(g, carry):
            dg = []
            for b in range(2):
                k = g * 2 + b
                off = pl.multiple_of(ebase + k * CH, CH)
                pltpu.make_async_copy(src.at[pl.ds(off, CH)], idxb[b],
                                      li[b]).wait()

                @pl.when(k >= 2)
                def _():
                    pltpu.make_async_copy(rowb[b], xg.at[pl.ds(off, CH)],
                                          w[b]).wait()

                dg.append(pltpu.async_copy(x16.at[idxb[b]], rowb[b], q[b]))
            for b in range(2):
                k = g * 2 + b
                off = pl.multiple_of(ebase + k * CH, CH)
                dg[b].wait()
                pltpu.async_copy(rowb[b], xg.at[pl.ds(off, CH)], w[b])

                @pl.when(k + 2 < nch)
                def _():
                    noff = pl.multiple_of(ebase + (k + 2) * CH, CH)
                    pltpu.async_copy(src.at[pl.ds(noff, CH)], idxb[b], li[b])
            return carry

        lax.fori_loop(0, nch // 2, rnd, 0)
        for b in range(2):
            pltpu.make_async_copy(rowb[b], xg.at[pl.ds(ebase, CH)],
                                  w[b]).wait()

    return body


def _ph_scatter(rows_src, idx_src, acc, ia, ra, li, lr, sc, ebase, nch):
    nb = len(ra)
    for b in range(nb):
        off = pl.multiple_of(ebase + b * CH, CH)
        pltpu.async_copy(idx_src.at[pl.ds(off, CH)], ia[b], li[b])
        pltpu.async_copy(rows_src.at[pl.ds(off, CH)], ra[b], lr[b])

    def rnd(g, carry):
        for half in range(2):
            dsc = []
            for j in range(2):
                b = half * 2 + j
                k = g * 4 + b
                off = pl.multiple_of(ebase + k * CH, CH)
                pltpu.make_async_copy(idx_src.at[pl.ds(off, CH)], ia[b],
                                      li[b]).wait()
                pltpu.make_async_copy(rows_src.at[pl.ds(off, CH)], ra[b],
                                      lr[b]).wait()
                dsc.append(pltpu.async_copy(ra[b], acc.at[ia[b]], sc,
                                            add=True))
            for j in range(2):
                b = half * 2 + j
                k = g * 4 + b
                dsc[j].wait()

                @pl.when(k + nb < nch)
                def _():
                    noff = pl.multiple_of(ebase + (k + nb) * CH, CH)
                    pltpu.async_copy(idx_src.at[pl.ds(noff, CH)], ia[b],
                                     li[b])
                    pltpu.async_copy(rows_src.at[pl.ds(noff, CH)], ra[b],
                                     lr[b])
        return carry

    lax.fori_loop(0, nch // 4, rnd, 0)


@functools.cache
def _sc_depth():
    @functools.partial(
        pl.kernel, mesh=_mesh(),
        out_type=[jax.ShapeDtypeStruct((EP, HH), f32) for _ in range(4)],
        scratch_types=[pltpu.VMEM_SHARED((NP, HH), f32)]
        + [pltpu.VMEM((CH,), i32)] * 8 + [pltpu.VMEM((CH, HH), f32)] * 4
        + [pltpu.SemaphoreType.DMA] * 17,
    )
    def body(m_a, m_b, dst, src, rev, zrows,
             g1_a, g1_b, g2_a, g2_b,
             acc, ia0, ia1, ia2, ia3, ia4, ia5, ia6, ia7,
             ra0, ra1, ra2, ra3,
             li0, li1, li2, li3, lr0, lr1, lr2, lr3, sc,
             q10, q11, q20, q21, w10, w11, w20, w21):
        c = lax.axis_index("c")
        s = lax.axis_index("s")
        ebase = s * EPT
        nch = EPT // CH
        ia = (ia0, ia1, ia2, ia3)
        iar = (ia4, ia5, ia6, ia7)
        ra = (ra0, ra1, ra2, ra3)
        li = (li0, li1, li2, li3)
        lr = (lr0, lr1, lr2, lr3)
        qs = (q10, q11, q20, q21)
        ws = (w10, w11, w20, w21)

        def core(m_h, g1_h, g2_h):
            pltpu.sync_copy(zrows, acc.at[pl.ds(s * NPT, NPT)])
            plsc.subcore_barrier()
            _ph_scatter(m_h, dst, acc, ia, ra, li, lr, sc, ebase, nch)
            for j in range(4):
                off = pl.multiple_of(ebase + j * CH, CH)
                pltpu.async_copy(src.at[pl.ds(off, CH)], ia[j], li[j])
                pltpu.async_copy(rev.at[pl.ds(off, CH)], iar[j], lr[j])
            plsc.subcore_barrier()

            def ph_c(g, carry):
                d2 = []
                for j in range(4):
                    k = g * 4 + j
                    off = pl.multiple_of(ebase + k * CH, CH)
                    pltpu.make_async_copy(src.at[pl.ds(off, CH)], ia[j],
                                          li[j]).wait()
                    pltpu.make_async_copy(rev.at[pl.ds(off, CH)], iar[j],
                                          lr[j]).wait()

                    @pl.when(k >= 4)
                    def _():
                        pltpu.make_async_copy(ra[j], g1_h.at[pl.ds(off, CH)],
                                              ws[j]).wait()

                    d2.append(pltpu.async_copy(m_h.at[iar[j]], ra[j], qs[j]))
                d1 = []
                for j in range(4):
                    k = g * 4 + j
                    off = pl.multiple_of(ebase + k * CH, CH)
                    d2[j].wait()
                    pltpu.async_copy(ra[j], g2_h.at[pl.ds(off, CH)], ws[j])
                for j in range(4):
                    k = g * 4 + j
                    off = pl.multiple_of(ebase + k * CH, CH)
                    pltpu.make_async_copy(ra[j], g2_h.at[pl.ds(off, CH)],
                                          ws[j]).wait()
                    d1.append(pltpu.async_copy(acc.at[ia[j]], ra[j], qs[j]))
                for j in range(4):
                    k = g * 4 + j
                    off = pl.multiple_of(ebase + k * CH, CH)
                    d1[j].wait()
                    pltpu.async_copy(ra[j], g1_h.at[pl.ds(off, CH)], ws[j])

                    @pl.when(k + 4 < nch)
                    def _():
                        noff = pl.multiple_of(ebase + (k + 4) * CH, CH)
                        pltpu.async_copy(src.at[pl.ds(noff, CH)], ia[j],
                                         li[j])
                        pltpu.async_copy(rev.at[pl.ds(noff, CH)], iar[j],
                                         lr[j])
                return carry

            lax.fori_loop(0, nch // 4, ph_c, 0)
            for j in range(4):
                pltpu.make_async_copy(ra[j], g1_h.at[pl.ds(ebase, CH)],
                                      ws[j]).wait()

        pl.when(c == 0)(lambda: core(m_a, g1_a, g2_a))
        pl.when(c == 1)(lambda: core(m_b, g1_b, g2_b))

    return body


@functools.cache
def _sc_segsum():
    @functools.partial(
        pl.kernel, mesh=_mesh(),
        out_type=[jax.ShapeDtypeStruct((NP, HH), f32) for _ in range(2)],
        scratch_types=[pltpu.VMEM_SHARED((NP, HH), f32)]
        + [pltpu.VMEM((CH,), i32)] * 4 + [pltpu.VMEM((CH, HH), f32)] * 4
        + [pltpu.SemaphoreType.DMA] * 9,
    )
    def body(h_a, h_b, dst, zrows, ns_a, ns_b,
             acc, ia0, ia1, ia2, ia3, ra0, ra1, ra2, ra3,
             li0, li1, li2, li3, lr0, lr1, lr2, lr3, sc):
        c = lax.axis_index("c")
        s = lax.axis_index("s")
        ebase = s * EPT
        ia, ra = (ia0, ia1, ia2, ia3), (ra0, ra1, ra2, ra3)
        li, lr = (li0, li1, li2, li3), (lr0, lr1, lr2, lr3)

        def core(h_h, ns_h):
            pltpu.sync_copy(zrows, acc.at[pl.ds(s * NPT, NPT)])
            plsc.subcore_barrier()
            _ph_scatter(h_h, dst, acc, ia, ra, li, lr, sc, ebase, EPT // CH)
            plsc.subcore_barrier()
            pltpu.sync_copy(acc.at[pl.ds(s * NPT, NPT)],
                            ns_h.at[pl.ds(s * NPT, NPT)])

        pl.when(c == 0)(lambda: core(h_a, ns_a))
        pl.when(c == 1)(lambda: core(h_b, ns_b))

    return body


@functools.cache
def _sc_pool():
    gpt = NG // NS

    nch = NPT // CH

    @functools.partial(
        pl.kernel, mesh=_mesh(),
        out_type=[jax.ShapeDtypeStruct((NG, HH), f32) for _ in range(2)],
        scratch_types=[pltpu.VMEM_SHARED((NG, HH), f32)]
        + [pltpu.VMEM((CH,), i32)] * 2 + [pltpu.VMEM((CH, HH), f32)] * 2
        + [pltpu.SemaphoreType.DMA] * 5,
    )
    def body(ne_a, ne_b, bat, zrows, p_a, p_b,
             acc, ia0, ia1, ra0, ra1, li0, li1, lr0, lr1, sc):
        c = lax.axis_index("c")
        s = lax.axis_index("s")
        nbase = s * NPT
        ia, ra, li, lr = (ia0, ia1), (ra0, ra1), (li0, li1), (lr0, lr1)

        def core(ne_h, p_h):
            dl = [None, None]
            dr = [None, None]
            for k in range(2):
                off = nbase + k * CH
                dl[k] = pltpu.async_copy(bat.at[pl.ds(off, CH)], ia[k], li[k])
                dr[k] = pltpu.async_copy(ne_h.at[pl.ds(off, CH)], ra[k],
                                         lr[k])
            pltpu.sync_copy(zrows.at[pl.ds(0, gpt)],
                            acc.at[pl.ds(s * gpt, gpt)])
            plsc.subcore_barrier()
            for k in range(nch):
                b = k % 2
                dl[b].wait()
                dr[b].wait()
                pltpu.async_copy(ra[b], acc.at[ia[b]], sc, add=True).wait()
                if k + 2 < nch:
                    off = nbase + (k + 2) * CH
                    dl[b] = pltpu.async_copy(bat.at[pl.ds(off, CH)], ia[b],
                                             li[b])
                    dr[b] = pltpu.async_copy(ne_h.at[pl.ds(off, CH)], ra[b],
                                             lr[b])
            plsc.subcore_barrier()
            pltpu.sync_copy(acc.at[pl.ds(s * gpt, gpt)],
                            p_h.at[pl.ds(s * gpt, gpt)])

        pl.when(c == 0)(lambda: core(ne_a, p_a))
        pl.when(c == 1)(lambda: core(ne_b, p_b))

    return body


BE = 4096


bf16 = jnp.bfloat16


def _tc_a_body(xg, ea, wi, wm, h0a, h0b, ma, mb):
    xe = xg[...] + jnp.pad(ea[...], ((0, 0), (6, 7)))
    h0 = _relu(jnp.dot(xe, wi[...], preferred_element_type=f32))
    m = _relu(jnp.dot(h0.astype(bf16), wm[...].astype(bf16),
                      preferred_element_type=f32))
    h0a[...] = h0[:, :HH].astype(bf16)
    h0b[...] = h0[:, HH:].astype(bf16)
    ma[...] = m[:, :HH]
    mb[...] = m[:, HH:]


def _tc_a(xg, ea, wi16, wm):
    eb = lambda i: (i, 0)
    return pl.pallas_call(
        _tc_a_body,
        grid=(EP // BE,),
        in_specs=[pl.BlockSpec((BE, 16), eb), pl.BlockSpec((BE, 3), eb),
                  pl.BlockSpec((16, H), lambda i: (0, 0)),
                  pl.BlockSpec((H, H), lambda i: (0, 0))],
        out_specs=[pl.BlockSpec((BE, HH), eb)] * 4,
        out_shape=[jax.ShapeDtypeStruct((EP, HH), bf16)] * 2
        + [jax.ShapeDtypeStruct((EP, HH), f32)] * 2,
    )(xg, ea, wi16, wm)


def _tc_b_body(h0a, h0b, g1a, g1b, g2a, g2b, wm, ma, mb):
    ha = h0a[...].astype(f32) + g1a[...] - g2a[...]
    hb = h0b[...].astype(f32) + g1b[...] - g2b[...]
    h = jnp.concatenate([ha, hb], axis=1)
    m = _relu(jnp.dot(h.astype(bf16), wm[...].astype(bf16),
                      preferred_element_type=f32))
    ma[...] = m[:, :HH]
    mb[...] = m[:, HH:]


def _tc_b(h0a, h0b, g1a, g1b, g2a, g2b, wm):
    eb = lambda i: (i, 0)
    return pl.pallas_call(
        _tc_b_body,
        grid=(EP // BE,),
        in_specs=[pl.BlockSpec((BE, HH), eb)] * 6
        + [pl.BlockSpec((H, H), lambda i: (0, 0))],
        out_specs=[pl.BlockSpec((BE, HH), eb)] * 2,
        out_shape=[jax.ShapeDtypeStruct((EP, HH), f32)] * 2,
    )(h0a, h0b, g1a, g1b, g2a, g2b, wm)


def _tc_c2_body(h0a, h0b, g1a, g1b, g2a, g2b, h3a, h3b):
    h3a[...] = h0a[...].astype(f32) + g1a[...] - g2a[...]
    h3b[...] = h0b[...].astype(f32) + g1b[...] - g2b[...]


def _tc_c2(h0a, h0b, g1a, g1b, g2a, g2b):
    eb = lambda i: (i, 0)
    return pl.pallas_call(
        _tc_c2_body,
        grid=(EP // BE,),
        in_specs=[pl.BlockSpec((BE, HH), eb)] * 6,
        out_specs=[pl.BlockSpec((BE, HH), eb)] * 2,
        out_shape=[jax.ShapeDtypeStruct((EP, HH), f32)] * 2,
    )(h0a, h0b, g1a, g1b, g2a, g2b)


def _tc_c_body(xp, nsa, nsb, wax, wah, nea, neb):
    ns = jnp.concatenate([nsa[...], nsb[...]], axis=1)
    ne = _relu(jnp.dot(xp[...], wax[...], preferred_element_type=f32)
               + jnp.dot(ns, wah[...], preferred_element_type=f32))
    nea[...] = ne[:, :HH]
    neb[...] = ne[:, HH:]


def _tc_c(x16, nsa, nsb, wax16, wah):
    nb = lambda i: (i, 0)
    nbk = 2048
    return pl.pallas_call(
        _tc_c_body,
        grid=(NP // nbk,),
        in_specs=[pl.BlockSpec((nbk, 16), nb), pl.BlockSpec((nbk, HH), nb),
                  pl.BlockSpec((nbk, HH), nb),
                  pl.BlockSpec((16, H), lambda i: (0, 0)),
                  pl.BlockSpec((H, H), lambda i: (0, 0))],
        out_specs=[pl.BlockSpec((nbk, HH), nb)] * 2,
        out_shape=[jax.ShapeDtypeStruct((NP, HH), f32)] * 2,
    )(x16, nsa, nsb, wax16, wah)


def _tc_d_body(pa, pb, w1, b1, w2, b2, wl, bl, out):
    p = jnp.concatenate([pa[...], pb[...]], axis=1)
    f1 = _relu(jnp.dot(p, w1[...], preferred_element_type=f32) + b1[...])
    f2 = jnp.dot(f1, w2[...], preferred_element_type=f32) + b2[...]
    out[...] = jnp.dot(f2, wl[...], preferred_element_type=f32) + bl[...]


def _tc_d(pa, pb, w1, b1, w2, b2, wl, bl):
    return pl.pallas_call(
        _tc_d_body,
        out_shape=jax.ShapeDtypeStruct((NG, 128), f32),
    )(pa, pb, w1, b1, w2, b2, wl, bl)


def kernel(x, edge_index, edge_attr, rev_edge, batch, depth,
           Wi, Wm, Wa, W1, b1, W2, b2, Wl, bl):
    src = edge_index[0].astype(i32)
    dst = edge_index[1].astype(i32)
    rev = rev_edge.astype(i32)
    bat = batch.astype(i32)

    padi = jnp.full((EP - NE,), NN, dtype=i32)
    src_p = jnp.concatenate([src, padi])
    dst_p = jnp.concatenate([dst, padi])
    rev_p = jnp.concatenate([rev, jnp.arange(NE, EP, dtype=i32)])
    ea_p = jnp.zeros((EP, 3), f32).at[:NE].set(edge_attr)
    x16 = jnp.zeros((NP, 16), f32).at[:NN, :6].set(x)
    bat_p = jnp.zeros((NP,), i32).at[:NN].set(bat)
    zrows = jnp.zeros((NPT, HH), f32)

    wi16 = jnp.zeros((16, H), f32).at[:9].set(Wi)
    wax16 = jnp.zeros((16, H), f32).at[:6].set(Wa[:6])
    wah = Wa[6:]

    (xg,) = _sc_gather_x()(x16, src_p)
    h0a, h0b, ma, mb = _tc_a(xg, ea_p, wi16, Wm)

    g1a = g1b = g2a = g2b = None
    for i in range(3):
        g1a, g1b, g2a, g2b = _sc_depth()(ma, mb, dst_p, src_p, rev_p, zrows)
        if i < 2:
            ma, mb = _tc_b(h0a, h0b, g1a, g1b, g2a, g2b, Wm)

    h3a, h3b = _tc_c2(h0a, h0b, g1a, g1b, g2a, g2b)
    nsa, nsb = _sc_segsum()(h3a, h3b, dst_p, zrows)
    nea, neb = _tc_c(x16, nsa, nsb, wax16, wah)
    pa, pb = _sc_pool()(nea, neb, bat_p, zrows)
    return _tc_d(pa, pb, W1, b1.reshape(1, -1), W2, b2.reshape(1, -1),
                 Wl, bl.reshape(1, -1))


# speedup vs baseline: 1.7360x; 1.0012x over previous
import functools

import jax
import jax.numpy as jnp
from jax import lax
from jax.experimental import pallas as pl
from jax.experimental.pallas import tpu as pltpu
from jax.experimental.pallas import tpu_sc as plsc

NN = 10000
NP = 10240
NE = 160000
EP = 163840
H = 256
HH = 128
NG = 256
NC, NS, CH = 2, 16, 64
EPT = EP // NS
EPW = EP // (NC * NS)
NPT = NP // NS

f32 = jnp.float32
i32 = jnp.int32


@functools.cache
def _mesh():
    return plsc.VectorSubcoreMesh(core_axis_name="c", subcore_axis_name="s",
                                  num_cores=NC, num_subcores=NS)


def _relu(v):
    return jnp.maximum(v, 0.0)


@functools.cache
def _sc_gather_x():
    @functools.partial(
        pl.kernel, mesh=_mesh(),
        out_type=[jax.ShapeDtypeStruct((EP, 16), f32)],
        scratch_types=[pltpu.VMEM((128,), i32)] * 4
        + [pltpu.VMEM((128, 16), f32)] * 4
        + [pltpu.SemaphoreType.DMA] * 12,
        compiler_params=pltpu.CompilerParams(use_tc_tiling_on_sc=False),
    )
    def body(x16, src, xg, idx0, idx1, idx2, idx3, row0, row1, row2, row3,
             li0, li1, li2, li3, q0, q1, q2, q3, w0, w1, w2, w3):
        c = lax.axis_index("c")
        s = lax.axis_index("s")
        ebase = (s * NC + c) * EPW
        idxb = (idx0, idx1, idx2, idx3)
        rowb = (row0, row1, row2, row3)
        li, q, w = (li0, li1, li2, li3), (q0, q1, q2, q3), (w0, w1, w2, w3)
        CH = 128
        nch = EPW // CH

        for b in range(4):
            pltpu.async_copy(src.at[pl.ds(ebase + b * CH, CH)], idxb[b], li[b])

        def rnd(g, carry):
            dg = []
            for b in range(4):
                k = g * 4 + b
                off = pl.multiple_of(ebase + k * CH, CH)
                pltpu.make_async_copy(src.at[pl.ds(off, CH)], idxb[b],
                                      li[b]).wait()

                @pl.when(k >= 4)
                def _():
                    pltpu.make_async_copy(rowb[b], xg.at[pl.ds(off, CH)],
                                          w[b]).wait()

                dg.append(pltpu.async_copy(x16.at[idxb[b]], rowb[b], q[b]))
            for b in range(4):
                k = g * 4 + b
                off = pl.multiple_of(ebase + k * CH, CH)
                dg[b].wait()
                pltpu.async_copy(rowb[b], xg.at[pl.ds(off, CH)], w[b])

                @pl.when(k + 4 < nch)
                def _():
                    noff = pl.multiple_of(ebase + (k + 4) * CH, CH)
                    pltpu.async_copy(src.at[pl.ds(noff, CH)], idxb[b], li[b])
            return carry

        lax.fori_loop(0, nch // 4, rnd, 0)
        for b in range(4):
            pltpu.make_async_copy(rowb[b], xg.at[pl.ds(ebase, CH)],
                                  w[b]).wait()

    return body


def _ph_scatter(rows_src, idx_src, acc, ia, ra, li, lr, sc, ebase, nch):
    nb = len(ra)
    for b in range(nb):
        off = pl.multiple_of(ebase + b * CH, CH)
        pltpu.async_copy(idx_src.at[pl.ds(off, CH)], ia[b], li[b])
        pltpu.async_copy(rows_src.at[pl.ds(off, CH)], ra[b], lr[b])

    def rnd(g, carry):
        for half in range(2):
            dsc = []
            for j in range(2):
                b = half * 2 + j
                k = g * 4 + b
                off = pl.multiple_of(ebase + k * CH, CH)
                pltpu.make_async_copy(idx_src.at[pl.ds(off, CH)], ia[b],
                                      li[b]).wait()
                pltpu.make_async_copy(rows_src.at[pl.ds(off, CH)], ra[b],
                                      lr[b]).wait()
                dsc.append(pltpu.async_copy(ra[b], acc.at[ia[b]], sc,
                                            add=True))
            for j in range(2):
                b = half * 2 + j
                k = g * 4 + b
                dsc[j].wait()

                @pl.when(k + nb < nch)
                def _():
                    noff = pl.multiple_of(ebase + (k + nb) * CH, CH)
                    pltpu.async_copy(idx_src.at[pl.ds(noff, CH)], ia[b],
                                     li[b])
                    pltpu.async_copy(rows_src.at[pl.ds(noff, CH)], ra[b],
                                     lr[b])
        return carry

    lax.fori_loop(0, nch // 4, rnd, 0)


@functools.cache
def _sc_depth():
    @functools.partial(
        pl.kernel, mesh=_mesh(),
        out_type=[jax.ShapeDtypeStruct((EP, HH), f32) for _ in range(4)],
        scratch_types=[pltpu.VMEM_SHARED((NP, HH), f32)]
        + [pltpu.VMEM((CH,), i32)] * 8 + [pltpu.VMEM((CH, HH), f32)] * 4
        + [pltpu.SemaphoreType.DMA] * 17,
    )
    def body(m_a, m_b, dst, src, rev, zrows,
             g1_a, g1_b, g2_a, g2_b,
             acc, ia0, ia1, ia2, ia3, ia4, ia5, ia6, ia7,
             ra0, ra1, ra2, ra3,
             li0, li1, li2, li3, lr0, lr1, lr2, lr3, sc,
             q10, q11, q20, q21, w10, w11, w20, w21):
        c = lax.axis_index("c")
        s = lax.axis_index("s")
        ebase = s * EPT
        nch = EPT // CH
        ia = (ia0, ia1, ia2, ia3)
        iar = (ia4, ia5, ia6, ia7)
        ra = (ra0, ra1, ra2, ra3)
        li = (li0, li1, li2, li3)
        lr = (lr0, lr1, lr2, lr3)
        qs = (q10, q11, q20, q21)
        ws = (w10, w11, w20, w21)

        def core(m_h, g1_h, g2_h):
            pltpu.sync_copy(zrows, acc.at[pl.ds(s * NPT, NPT)])
            plsc.subcore_barrier()
            _ph_scatter(m_h, dst, acc, ia, ra, li, lr, sc, ebase, nch)
            for j in range(4):
                off = pl.multiple_of(ebase + j * CH, CH)
                pltpu.async_copy(src.at[pl.ds(off, CH)], ia[j], li[j])
                pltpu.async_copy(rev.at[pl.ds(off, CH)], iar[j], lr[j])
            plsc.subcore_barrier()

            def ph_c(g, carry):
                d2 = []
                for j in range(4):
                    k = g * 4 + j
                    off = pl.multiple_of(ebase + k * CH, CH)
                    pltpu.make_async_copy(src.at[pl.ds(off, CH)], ia[j],
                                          li[j]).wait()
                    pltpu.make_async_copy(rev.at[pl.ds(off, CH)], iar[j],
                                          lr[j]).wait()

                    @pl.when(k >= 4)
                    def _():
                        pltpu.make_async_copy(ra[j], g1_h.at[pl.ds(off, CH)],
                                              ws[j]).wait()

                    d2.append(pltpu.async_copy(m_h.at[iar[j]], ra[j], qs[j]))
                d1 = []
                for j in range(4):
                    k = g * 4 + j
                    off = pl.multiple_of(ebase + k * CH, CH)
                    d2[j].wait()
                    pltpu.async_copy(ra[j], g2_h.at[pl.ds(off, CH)], ws[j])
                for j in range(4):
                    k = g * 4 + j
                    off = pl.multiple_of(ebase + k * CH, CH)
                    pltpu.make_async_copy(ra[j], g2_h.at[pl.ds(off, CH)],
                                          ws[j]).wait()
                    d1.append(pltpu.async_copy(acc.at[ia[j]], ra[j], qs[j]))
                for j in range(4):
                    k = g * 4 + j
                    off = pl.multiple_of(ebase + k * CH, CH)
                    d1[j].wait()
                    pltpu.async_copy(ra[j], g1_h.at[pl.ds(off, CH)], ws[j])

                    @pl.when(k + 4 < nch)
                    def _():
                        noff = pl.multiple_of(ebase + (k + 4) * CH, CH)
                        pltpu.async_copy(src.at[pl.ds(noff, CH)], ia[j],
                                         li[j])
                        pltpu.async_copy(rev.at[pl.ds(noff, CH)], iar[j],
                                         lr[j])
                return carry

            lax.fori_loop(0, nch // 4, ph_c, 0)
            for j in range(4):
                pltpu.make_async_copy(ra[j], g1_h.at[pl.ds(ebase, CH)],
                                      ws[j]).wait()

        pl.when(c == 0)(lambda: core(m_a, g1_a, g2_a))
        pl.when(c == 1)(lambda: core(m_b, g1_b, g2_b))

    return body


@functools.cache
def _sc_segsum():
    @functools.partial(
        pl.kernel, mesh=_mesh(),
        out_type=[jax.ShapeDtypeStruct((NP, HH), f32) for _ in range(2)],
        scratch_types=[pltpu.VMEM_SHARED((NP, HH), f32)]
        + [pltpu.VMEM((CH,), i32)] * 4 + [pltpu.VMEM((CH, HH), f32)] * 4
        + [pltpu.SemaphoreType.DMA] * 9,
    )
    def body(h_a, h_b, dst, zrows, ns_a, ns_b,
             acc, ia0, ia1, ia2, ia3, ra0, ra1, ra2, ra3,
             li0, li1, li2, li3, lr0, lr1, lr2, lr3, sc):
        c = lax.axis_index("c")
        s = lax.axis_index("s")
        ebase = s * EPT
        ia, ra = (ia0, ia1, ia2, ia3), (ra0, ra1, ra2, ra3)
        li, lr = (li0, li1, li2, li3), (lr0, lr1, lr2, lr3)

        def core(h_h, ns_h):
            pltpu.sync_copy(zrows, acc.at[pl.ds(s * NPT, NPT)])
            plsc.subcore_barrier()
            _ph_scatter(h_h, dst, acc, ia, ra, li, lr, sc, ebase, EPT // CH)
            plsc.subcore_barrier()
            pltpu.sync_copy(acc.at[pl.ds(s * NPT, NPT)],
                            ns_h.at[pl.ds(s * NPT, NPT)])

        pl.when(c == 0)(lambda: core(h_a, ns_a))
        pl.when(c == 1)(lambda: core(h_b, ns_b))

    return body


@functools.cache
def _sc_pool():
    gpt = NG // NS

    nch = NPT // CH

    @functools.partial(
        pl.kernel, mesh=_mesh(),
        out_type=[jax.ShapeDtypeStruct((NG, HH), f32) for _ in range(2)],
        scratch_types=[pltpu.VMEM_SHARED((NG, HH), f32)]
        + [pltpu.VMEM((CH,), i32)] * 2 + [pltpu.VMEM((CH, HH), f32)] * 2
        + [pltpu.SemaphoreType.DMA] * 5,
    )
    def body(ne_a, ne_b, bat, zrows, p_a, p_b,
             acc, ia0, ia1, ra0, ra1, li0, li1, lr0, lr1, sc):
        c = lax.axis_index("c")
        s = lax.axis_index("s")
        nbase = s * NPT
        ia, ra, li, lr = (ia0, ia1), (ra0, ra1), (li0, li1), (lr0, lr1)

        def core(ne_h, p_h):
            dl = [None, None]
            dr = [None, None]
            for k in range(2):
                off = nbase + k * CH
                dl[k] = pltpu.async_copy(bat.at[pl.ds(off, CH)], ia[k], li[k])
                dr[k] = pltpu.async_copy(ne_h.at[pl.ds(off, CH)], ra[k],
                                         lr[k])
            pltpu.sync_copy(zrows.at[pl.ds(0, gpt)],
                            acc.at[pl.ds(s * gpt, gpt)])
            plsc.subcore_barrier()
            for k in range(nch):
                b = k % 2
                dl[b].wait()
                dr[b].wait()
                pltpu.async_copy(ra[b], acc.at[ia[b]], sc, add=True).wait()
                if k + 2 < nch:
                    off = nbase + (k + 2) * CH
                    dl[b] = pltpu.async_copy(bat.at[pl.ds(off, CH)], ia[b],
                                             li[b])
                    dr[b] = pltpu.async_copy(ne_h.at[pl.ds(off, CH)], ra[b],
                                             lr[b])
            plsc.subcore_barrier()
            pltpu.sync_copy(acc.at[pl.ds(s * gpt, gpt)],
                            p_h.at[pl.ds(s * gpt, gpt)])

        pl.when(c == 0)(lambda: core(ne_a, p_a))
        pl.when(c == 1)(lambda: core(ne_b, p_b))

    return body


BE = 4096


bf16 = jnp.bfloat16


def _tc_a_body(xg, ea, wi, wm, h0a, h0b, ma, mb):
    xe = xg[...] + jnp.pad(ea[...], ((0, 0), (6, 7)))
    h0 = _relu(jnp.dot(xe, wi[...], preferred_element_type=f32))
    m = _relu(jnp.dot(h0.astype(bf16), wm[...].astype(bf16),
                      preferred_element_type=f32))
    h0a[...] = h0[:, :HH].astype(bf16)
    h0b[...] = h0[:, HH:].astype(bf16)
    ma[...] = m[:, :HH]
    mb[...] = m[:, HH:]


def _tc_a(xg, ea, wi16, wm):
    eb = lambda i: (i, 0)
    return pl.pallas_call(
        _tc_a_body,
        grid=(EP // BE,),
        in_specs=[pl.BlockSpec((BE, 16), eb), pl.BlockSpec((BE, 3), eb),
                  pl.BlockSpec((16, H), lambda i: (0, 0)),
                  pl.BlockSpec((H, H), lambda i: (0, 0))],
        out_specs=[pl.BlockSpec((BE, HH), eb)] * 4,
        out_shape=[jax.ShapeDtypeStruct((EP, HH), bf16)] * 2
        + [jax.ShapeDtypeStruct((EP, HH), f32)] * 2,
    )(xg, ea, wi16, wm)


def _tc_b_body(h0a, h0b, g1a, g1b, g2a, g2b, wm, ma, mb):
    ha = h0a[...].astype(f32) + g1a[...] - g2a[...]
    hb = h0b[...].astype(f32) + g1b[...] - g2b[...]
    h = jnp.concatenate([ha, hb], axis=1)
    m = _relu(jnp.dot(h.astype(bf16), wm[...].astype(bf16),
                      preferred_element_type=f32))
    ma[...] = m[:, :HH]
    mb[...] = m[:, HH:]


def _tc_b(h0a, h0b, g1a, g1b, g2a, g2b, wm):
    eb = lambda i: (i, 0)
    return pl.pallas_call(
        _tc_b_body,
        grid=(EP // BE,),
        in_specs=[pl.BlockSpec((BE, HH), eb)] * 6
        + [pl.BlockSpec((H, H), lambda i: (0, 0))],
        out_specs=[pl.BlockSpec((BE, HH), eb)] * 2,
        out_shape=[jax.ShapeDtypeStruct((EP, HH), f32)] * 2,
    )(h0a, h0b, g1a, g1b, g2a, g2b, wm)


def _tc_c2_body(h0a, h0b, g1a, g1b, g2a, g2b, h3a, h3b):
    h3a[...] = h0a[...].astype(f32) + g1a[...] - g2a[...]
    h3b[...] = h0b[...].astype(f32) + g1b[...] - g2b[...]


def _tc_c2(h0a, h0b, g1a, g1b, g2a, g2b):
    eb = lambda i: (i, 0)
    return pl.pallas_call(
        _tc_c2_body,
        grid=(EP // BE,),
        in_specs=[pl.BlockSpec((BE, HH), eb)] * 6,
        out_specs=[pl.BlockSpec((BE, HH), eb)] * 2,
        out_shape=[jax.ShapeDtypeStruct((EP, HH), f32)] * 2,
    )(h0a, h0b, g1a, g1b, g2a, g2b)


def _tc_c_body(xp, nsa, nsb, wax, wah, nea, neb):
    ns = jnp.concatenate([nsa[...], nsb[...]], axis=1)
    ne = _relu(jnp.dot(xp[...], wax[...], preferred_element_type=f32)
               + jnp.dot(ns, wah[...], preferred_element_type=f32))
    nea[...] = ne[:, :HH]
    neb[...] = ne[:, HH:]


def _tc_c(x16, nsa, nsb, wax16, wah):
    nb = lambda i: (i, 0)
    nbk = 2048
    return pl.pallas_call(
        _tc_c_body,
        grid=(NP // nbk,),
        in_specs=[pl.BlockSpec((nbk, 16), nb), pl.BlockSpec((nbk, HH), nb),
                  pl.BlockSpec((nbk, HH), nb),
                  pl.BlockSpec((16, H), lambda i: (0, 0)),
                  pl.BlockSpec((H, H), lambda i: (0, 0))],
        out_specs=[pl.BlockSpec((nbk, HH), nb)] * 2,
        out_shape=[jax.ShapeDtypeStruct((NP, HH), f32)] * 2,
    )(x16, nsa, nsb, wax16, wah)


def _tc_d_body(pa, pb, w1, b1, w2, b2, wl, bl, out):
    p = jnp.concatenate([pa[...], pb[...]], axis=1)
    f1 = _relu(jnp.dot(p, w1[...], preferred_element_type=f32) + b1[...])
    f2 = jnp.dot(f1, w2[...], preferred_element_type=f32) + b2[...]
    out[...] = jnp.dot(f2, wl[...], preferred_element_type=f32) + bl[...]


def _tc_d(pa, pb, w1, b1, w2, b2, wl, bl):
    return pl.pallas_call(
        _tc_d_body,
        out_shape=jax.ShapeDtypeStruct((NG, 128), f32),
    )(pa, pb, w1, b1, w2, b2, wl, bl)


def kernel(x, edge_index, edge_attr, rev_edge, batch, depth,
           Wi, Wm, Wa, W1, b1, W2, b2, Wl, bl):
    src = edge_index[0].astype(i32)
    dst = edge_index[1].astype(i32)
    rev = rev_edge.astype(i32)
    bat = batch.astype(i32)

    padi = jnp.full((EP - NE,), NN, dtype=i32)
    src_p = jnp.concatenate([src, padi])
    dst_p = jnp.concatenate([dst, padi])
    rev_p = jnp.concatenate([rev, jnp.arange(NE, EP, dtype=i32)])
    ea_p = jnp.zeros((EP, 3), f32).at[:NE].set(edge_attr)
    x16 = jnp.zeros((NP, 16), f32).at[:NN, :6].set(x)
    bat_p = jnp.zeros((NP,), i32).at[:NN].set(bat)
    zrows = jnp.zeros((NPT, HH), f32)

    wi16 = jnp.zeros((16, H), f32).at[:9].set(Wi)
    wax16 = jnp.zeros((16, H), f32).at[:6].set(Wa[:6])
    wah = Wa[6:]

    (xg,) = _sc_gather_x()(x16, src_p)
    h0a, h0b, ma, mb = _tc_a(xg, ea_p, wi16, Wm)

    g1a = g1b = g2a = g2b = None
    for i in range(3):
        g1a, g1b, g2a, g2b = _sc_depth()(ma, mb, dst_p, src_p, rev_p, zrows)
        if i < 2:
            ma, mb = _tc_b(h0a, h0b, g1a, g1b, g2a, g2b, Wm)

    h3a, h3b = _tc_c2(h0a, h0b, g1a, g1b, g2a, g2b)
    nsa, nsb = _sc_segsum()(h3a, h3b, dst_p, zrows)
    nea, neb = _tc_c(x16, nsa, nsb, wax16, wah)
    pa, pb = _sc_pool()(nea, neb, bat_p, zrows)
    return _tc_d(pa, pb, W1, b1.reshape(1, -1), W2, b2.reshape(1, -1),
                 Wl, bl.reshape(1, -1))
